# Initial kernel scaffold; baseline (speedup 1.0000x reference)
import jax
import jax.numpy as jnp
from jax.experimental import pallas as pl


def kernel(inputs, edge_index, w, a1_w, a1_b, bn1_g, bn1_b, a2_w, a2_b, bn2_g, bn2_b, a3_w, a3_b):
    raise NotImplementedError("write your pallas kernel here")


# speedup vs baseline: 1.2965x; 1.2965x over previous
import functools

import jax
import jax.numpy as jnp
from jax import lax
from jax.experimental import pallas as pl
from jax.experimental.pallas import tpu as pltpu
from jax.experimental.pallas import tpu_sc as plsc

NC = 2
NS = 16
NW = NC * NS
LRELU_SLOPE = 0.2
ACC_W = 144


def _lrelu(x):
    return jnp.where(x > 0, x, LRELU_SLOPE * x)


def _matmul_h(inputs, w):
    n, d_in = inputs.shape
    d_out = w.shape[1]
    hf = d_out // 2

    def body(x_ref, w_ref, o_ref, o0_ref, o1_ref):
        o = jnp.dot(x_ref[...], w_ref[...], preferred_element_type=jnp.float32)
        o_ref[...] = o
        o0_ref[...] = o[:, :hf]
        o1_ref[...] = o[:, hf:]

    return pl.pallas_call(
        body,
        out_shape=[
            jax.ShapeDtypeStruct((n, d_out), jnp.float32),
            jax.ShapeDtypeStruct((n, hf), jnp.float32),
            jax.ShapeDtypeStruct((n, hf), jnp.float32),
        ],
    )(inputs, w)


def _sc_gather_absdiff(h, src, dst):
    n, d = h.shape
    e = src.shape[0]
    tpe = e // NW
    b = 80
    chunks = tpe // b
    nv = d // 16
    mesh = plsc.VectorSubcoreMesh(core_axis_name="c", subcore_axis_name="s")

    @functools.partial(
        pl.kernel,
        out_type=jax.ShapeDtypeStruct((e, d), jnp.float32),
        mesh=mesh,
        scratch_types=[
            pltpu.VMEM((b,), jnp.int32),
            pltpu.VMEM((b,), jnp.int32),
            pltpu.VMEM((b, d), jnp.float32),
            pltpu.VMEM((b, d), jnp.float32),
            pltpu.SemaphoreType.DMA,
            pltpu.SemaphoreType.DMA,
        ],
    )
    def k(h_hbm, src_hbm, dst_hbm, out_hbm, si_v, di_v, a_v, b_v, sem_a, sem_b):
        wid = lax.axis_index("s") * NC + lax.axis_index("c")
        base0 = wid * tpe

        @pl.loop(0, chunks)
        def _chunk(i):
            base = base0 + i * b
            pltpu.sync_copy(src_hbm.at[pl.ds(base, b)], si_v)
            pltpu.sync_copy(dst_hbm.at[pl.ds(base, b)], di_v)
            cpa = pltpu.async_copy(h_hbm.at[si_v], a_v, sem_a)
            cpb = pltpu.async_copy(h_hbm.at[di_v], b_v, sem_b)
            cpa.wait()
            cpb.wait()

            @pl.loop(0, b)
            def _row(j):
                for kk in range(nv):
                    sl = pl.ds(kk * 16, 16)
                    a_v[j, sl] = jnp.abs(a_v[j, sl] - b_v[j, sl])

            pltpu.sync_copy(a_v, out_hbm.at[pl.ds(base, b)])

    return k(h, src, dst)


def _stage1(edge_h, a1_w):
    e, d = edge_h.shape
    f = a1_w.shape[1]
    be = 2560
    g = e // be

    def body(eh_ref, w_ref, x1_ref, st_ref):
        x1 = jnp.dot(eh_ref[...], w_ref[...], preferred_element_type=jnp.float32)
        x1_ref[...] = x1

        @pl.when(pl.program_id(0) == 0)
        def _():
            st_ref[...] = jnp.zeros_like(st_ref)

        st_ref[...] += jnp.stack(
            [jnp.sum(x1, axis=0), jnp.sum(x1 * x1, axis=0)])

    return pl.pallas_call(
        body,
        grid=(g,),
        in_specs=[
            pl.BlockSpec((be, d), lambda i: (i, 0)),
            pl.BlockSpec((d, f), lambda i: (0, 0)),
        ],
        out_specs=[
            pl.BlockSpec((be, f), lambda i: (i, 0)),
            pl.BlockSpec((2, f), lambda i: (0, 0)),
        ],
        out_shape=[
            jax.ShapeDtypeStruct((e, f), jnp.float32),
            jax.ShapeDtypeStruct((2, f), jnp.float32),
        ],
    )(edge_h, a1_w)


def _stage2(x1, scale1, shift1, a2_w):
    e, f1 = x1.shape
    f2 = a2_w.shape[1]
    be = 2560
    g = e // be

    def body(x1_ref, sc_ref, sh_ref, w_ref, x2_ref, st_ref):
        y = _lrelu(x1_ref[...] * sc_ref[...] + sh_ref[...])
        x2 = jnp.dot(y, w_ref[...], preferred_element_type=jnp.float32)
        x2_ref[...] = x2

        @pl.when(pl.program_id(0) == 0)
        def _():
            st_ref[...] = jnp.zeros_like(st_ref)

        st_ref[...] += jnp.stack(
            [jnp.sum(x2, axis=0), jnp.sum(x2 * x2, axis=0)])

    return pl.pallas_call(
        body,
        grid=(g,),
        in_specs=[
            pl.BlockSpec((be, f1), lambda i: (i, 0)),
            pl.BlockSpec((1, f1), lambda i: (0, 0)),
            pl.BlockSpec((1, f1), lambda i: (0, 0)),
            pl.BlockSpec((f1, f2), lambda i: (0, 0)),
        ],
        out_specs=[
            pl.BlockSpec((be, f2), lambda i: (i, 0)),
            pl.BlockSpec((2, f2), lambda i: (0, 0)),
        ],
        out_shape=[
            jax.ShapeDtypeStruct((e, f2), jnp.float32),
            jax.ShapeDtypeStruct((2, f2), jnp.float32),
        ],
    )(x1, scale1, shift1, a2_w)


def _stage3(x2, scale2, shift2, a3_w_row, a3_b, srcc, dstc):
    e, f2 = x2.shape
    be = 2560
    g = e // be

    def body(x2_ref, sc_ref, sh_ref, w3_ref, b3_ref, s_ref, d_ref, o_ref):
        y = _lrelu(x2_ref[...] * sc_ref[...] + sh_ref[...])
        t = jnp.sum(y * w3_ref[...], axis=1, keepdims=True) + b3_ref[...]
        t = _lrelu(t)
        ee = jnp.exp(-t) + (s_ref[...] == d_ref[...]).astype(jnp.float32)
        o_ref[...] = ee

    return pl.pallas_call(
        body,
        grid=(g,),
        in_specs=[
            pl.BlockSpec((be, f2), lambda i: (i, 0)),
            pl.BlockSpec((1, f2), lambda i: (0, 0)),
            pl.BlockSpec((1, f2), lambda i: (0, 0)),
            pl.BlockSpec((1, f2), lambda i: (0, 0)),
            pl.BlockSpec((1, 1), lambda i: (0, 0)),
            pl.BlockSpec((be, 1), lambda i: (i, 0)),
            pl.BlockSpec((be, 1), lambda i: (i, 0)),
        ],
        out_specs=pl.BlockSpec((be, 1), lambda i: (i, 0)),
        out_shape=jax.ShapeDtypeStruct((e, 1), jnp.float32),
    )(x2, scale2, shift2, a3_w_row, a3_b, srcc, dstc)


def _sc_scatter(h0, h1, n, src, dst, edge_e):
    hf = h0.shape[1]
    e = src.shape[0]
    tpe = e // NS
    b = 80
    chunks = tpe // b
    nvh = hf // 16
    n_pad = ((n + 1023) // 1024) * 1024
    a_rows = n_pad // 2
    rows_per_tile = a_rows // NS
    mesh = plsc.VectorSubcoreMesh(core_axis_name="c", subcore_axis_name="s")

    @functools.partial(
        pl.kernel,
        out_type=jax.ShapeDtypeStruct((NC * a_rows, 2 * hf), jnp.float32),
        mesh=mesh,
        compiler_params=pltpu.CompilerParams(use_tc_tiling_on_sc=False),
        scratch_types=[
            pltpu.VMEM((b,), jnp.int32),
            pltpu.VMEM((b,), jnp.int32),
            pltpu.VMEM((b,), jnp.int32),
            pltpu.VMEM((b,), jnp.float32),
            pltpu.VMEM((b, hf), jnp.float32),
            pltpu.VMEM((b, 2 * hf), jnp.float32),
            pltpu.VMEM((rows_per_tile, 2 * hf), jnp.float32),
            pltpu.VMEM_SHARED((a_rows, 2 * hf), jnp.float32),
            pltpu.SemaphoreType.DMA,
        ],
    )
    def k(h0_hbm, h1_hbm, src_hbm, dst_hbm, e_hbm, out_hbm,
          si_v, si2_v, di_v, ev_v, rows_v, sc_v, bounce_v, acc_sh, sem):
        c = lax.axis_index("c")
        s = lax.axis_index("s")
        base0 = s * tpe

        @pl.loop(0, rows_per_tile)
        def _z(j):
            for kk in range(2 * nvh):
                bounce_v[j, pl.ds(kk * 16, 16)] = jnp.zeros((16,), jnp.float32)

        pltpu.sync_copy(bounce_v, acc_sh.at[pl.ds(s * rows_per_tile, rows_per_tile)])
        plsc.subcore_barrier()

        @pl.loop(0, chunks)
        def _chunk(i):
            base = base0 + i * b
            pltpu.sync_copy(src_hbm.at[pl.ds(base, b)], si_v)
            pltpu.sync_copy(dst_hbm.at[pl.ds(base, b)], di_v)
            pltpu.sync_copy(e_hbm.at[pl.ds(base, b)], ev_v)

            @pl.when(c == 0)
            def _g0():
                pltpu.async_copy(h0_hbm.at[di_v], rows_v, sem).wait()

            @pl.when(c == 1)
            def _g1():
                pltpu.async_copy(h1_hbm.at[di_v], rows_v, sem).wait()

            @pl.loop(0, b // 16)
            def _grp(gidx):
                evec = ev_v[pl.ds(gidx * 16, 16)]
                sivec = si_v[pl.ds(gidx * 16, 16)]
                si2_v[pl.ds(gidx * 16, 16)] = lax.shift_right_logical(sivec, 1)
                for jj in range(16):
                    j = gidx * 16 + jj
                    ee = evec[jj]
                    par = sivec[jj] & 1
                    e0 = jnp.where(par == 0, ee, 0.0)
                    e1 = ee - e0
                    for kk in range(nvh):
                        v = rows_v[j, pl.ds(kk * 16, 16)]
                        sc_v[j, pl.ds(kk * 16, 16)] = v * e0
                        sc_v[j, pl.ds(hf + kk * 16, 16)] = v * e1

            pltpu.sync_copy(sc_v, acc_sh.at[si2_v], add=True)

        plsc.subcore_barrier()
        pltpu.sync_copy(acc_sh.at[pl.ds(s * rows_per_tile, rows_per_tile)], bounce_v)
        pltpu.sync_copy(bounce_v,
                        out_hbm.at[pl.ds(c * a_rows + s * rows_per_tile,
                                         rows_per_tile)])

    return k(h0, h1, src, dst, edge_e), n_pad


def _sc_rowsum(n, src, edge_e):
    e = src.shape[0]
    tpe = e // NW
    b = 80
    chunks = tpe // b
    n_pad = ((n + 1023) // 1024) * 1024
    rs_rows = n_pad // 8
    rs_per_tile = rs_rows // NS
    mesh = plsc.VectorSubcoreMesh(core_axis_name="c", subcore_axis_name="s")

    @functools.partial(
        pl.kernel,
        out_type=jax.ShapeDtypeStruct((NC * rs_rows, 128), jnp.float32),
        mesh=mesh,
        scratch_types=[
            pltpu.VMEM((b,), jnp.int32),
            pltpu.VMEM((b,), jnp.int32),
            pltpu.VMEM((b,), jnp.float32),
            pltpu.VMEM((b, 128), jnp.float32),
            pltpu.VMEM((rs_per_tile, 128), jnp.float32),
            pltpu.VMEM_SHARED((rs_rows, 128), jnp.float32),
        ],
    )
    def k(src_hbm, e_hbm, ors_hbm, si_v, si8_v, ev_v, rs_src_v, bounce_v, rs_sh):
        c = lax.axis_index("c")
        s = lax.axis_index("s")
        wid = s * NC + c
        base0 = wid * tpe
        lane = lax.iota(jnp.int32, 16)

        @pl.loop(0, b)
        def _z2(j):
            for kk in range(8):
                rs_src_v[j, pl.ds(kk * 16, 16)] = jnp.zeros((16,), jnp.float32)

        @pl.loop(0, rs_per_tile)
        def _z3(j):
            for kk in range(8):
                bounce_v[j, pl.ds(kk * 16, 16)] = jnp.zeros((16,), jnp.float32)

        pltpu.sync_copy(bounce_v, rs_sh.at[pl.ds(s * rs_per_tile, rs_per_tile)])
        plsc.subcore_barrier()

        @pl.loop(0, chunks)
        def _chunk(i):
            base = base0 + i * b
            pltpu.sync_copy(src_hbm.at[pl.ds(base, b)], si_v)
            pltpu.sync_copy(e_hbm.at[pl.ds(base, b)], ev_v)

            @pl.loop(0, b // 16)
            def _grp(gidx):
                evec = ev_v[pl.ds(gidx * 16, 16)]
                sivec = si_v[pl.ds(gidx * 16, 16)]
                si8_v[pl.ds(gidx * 16, 16)] = lax.shift_right_logical(sivec, 3)
                for jj in range(16):
                    j = gidx * 16 + jj
                    r = sivec[jj] & 7
                    rs_src_v[j, pl.ds(0, 16)] = jnp.where(lane == r, evec[jj], 0.0)

            pltpu.sync_copy(rs_src_v, rs_sh.at[si8_v], add=True)

        plsc.subcore_barrier()
        pltpu.sync_copy(rs_sh.at[pl.ds(s * rs_per_tile, rs_per_tile)], bounce_v)
        pltpu.sync_copy(bounce_v,
                        ors_hbm.at[pl.ds(c * rs_rows + s * rs_per_tile,
                                         rs_per_tile)])

    return k(src, edge_e), rs_rows


def _finalize(a0, a1, rs_col, n, d):
    bn = 2000
    g = n // bn
    hf = d // 2

    def body(a0_ref, a1_ref, r_ref, o_ref):
        rs = r_ref[...]
        rs = jnp.where(rs == 0.0, 1.0, rs)
        o_ref[...] = _lrelu(
            jnp.concatenate([a0_ref[...], a1_ref[...]], axis=1) / rs)

    return pl.pallas_call(
        body,
        grid=(g,),
        in_specs=[
            pl.BlockSpec((bn, hf), lambda i: (i, 0)),
            pl.BlockSpec((bn, hf), lambda i: (i, 0)),
            pl.BlockSpec((bn, 1), lambda i: (i, 0)),
        ],
        out_specs=pl.BlockSpec((bn, d), lambda i: (i, 0)),
        out_shape=jax.ShapeDtypeStruct((n, d), jnp.float32),
    )(a0, a1, rs_col)


def _bn_affine(stats, gamma, beta, count, eps=1e-5):
    mean = stats[0] / count
    var = stats[1] / count - mean * mean
    inv = gamma / jnp.sqrt(var + eps)
    scale = inv
    shift = beta - mean * inv
    return scale.reshape(1, -1), shift.reshape(1, -1)


def kernel(inputs, edge_index, w, a1_w, a1_b, bn1_g, bn1_b,
           a2_w, a2_b, bn2_g, bn2_b, a3_w, a3_b):
    e = edge_index.shape[1]
    src = edge_index[0].astype(jnp.int32)
    dst = edge_index[1].astype(jnp.int32)

    h, h0, h1 = _matmul_h(inputs, w)
    edge_h = _sc_gather_absdiff(h, src, dst)

    x1, st1 = _stage1(edge_h, a1_w)
    scale1, shift1 = _bn_affine(st1, bn1_g, bn1_b, e)
    x2, st2 = _stage2(x1, scale1, shift1, a2_w)
    scale2, shift2 = _bn_affine(st2, bn2_g, bn2_b, e)
    edge_e = _stage3(x2, scale2, shift2, a3_w.reshape(1, -1),
                     a3_b.reshape(1, 1), src.reshape(e, 1), dst.reshape(e, 1))

    n, d = inputs.shape[0], w.shape[1]
    ef = edge_e.reshape(e)
    acc, n_pad = _sc_scatter(h0, h1, n, src, dst, ef)
    rs, rs_rows = _sc_rowsum(n, src, ef)
    a_rows = n_pad // 2
    a0 = acc[:a_rows].reshape(n_pad, d // 2)[:n]
    a1 = acc[a_rows:].reshape(n_pad, d // 2)[:n]
    rs_col = (rs[:rs_rows, :8] + rs[rs_rows:, :8]).reshape(n_pad)[:n].reshape(n, 1)
    return _finalize(a0, a1, rs_col, n, d)


# speedup vs baseline: 1.8320x; 1.4131x over previous
import functools

import jax
import jax.numpy as jnp
from jax import lax
from jax.experimental import pallas as pl
from jax.experimental.pallas import tpu as pltpu
from jax.experimental.pallas import tpu_sc as plsc

NC = 2
NS = 16
NW = NC * NS
LRELU_SLOPE = 0.2
ACC_W = 144


def _lrelu(x):
    return jnp.where(x > 0, x, LRELU_SLOPE * x)


def _matmul_h(inputs, w):
    n, d_in = inputs.shape
    d_out = w.shape[1]
    hf = d_out // 2

    def body(x_ref, w_ref, o_ref, o0_ref, o1_ref):
        o = jnp.dot(x_ref[...], w_ref[...], preferred_element_type=jnp.float32)
        o_ref[...] = o
        o0_ref[...] = o[:, :hf]
        o1_ref[...] = o[:, hf:]

    return pl.pallas_call(
        body,
        out_shape=[
            jax.ShapeDtypeStruct((n, d_out), jnp.float32),
            jax.ShapeDtypeStruct((n, hf), jnp.float32),
            jax.ShapeDtypeStruct((n, hf), jnp.float32),
        ],
    )(inputs, w)


def _sc_gather_absdiff(h, src, dst):
    n, d = h.shape
    e = src.shape[0]
    tpe = e // NW
    b = 80
    chunks = tpe // b
    nv = d // 16
    mesh = plsc.VectorSubcoreMesh(core_axis_name="c", subcore_axis_name="s")

    @functools.partial(
        pl.kernel,
        out_type=jax.ShapeDtypeStruct((e, d), jnp.float32),
        mesh=mesh,
        scratch_types=[
            pltpu.VMEM((2, b), jnp.int32),
            pltpu.VMEM((2, b), jnp.int32),
            pltpu.VMEM((2, b, d), jnp.float32),
            pltpu.VMEM((2, b, d), jnp.float32),
            [pltpu.SemaphoreType.DMA] * 2,
            [pltpu.SemaphoreType.DMA] * 2,
            [pltpu.SemaphoreType.DMA] * 2,
        ],
    )
    def k(h_hbm, src_hbm, dst_hbm, out_hbm, si_v, di_v, a_v, b_v,
          semi, semg, semo):
        wid = lax.axis_index("s") * NC + lax.axis_index("c")
        base0 = wid * tpe

        def idx_issue(j, sl):
            pltpu.async_copy(src_hbm.at[pl.ds(base0 + j * b, b)], si_v.at[sl], semi[sl])
            pltpu.async_copy(dst_hbm.at[pl.ds(base0 + j * b, b)], di_v.at[sl], semi[sl])

        def idx_wait(sl):
            pltpu.make_async_copy(src_hbm.at[pl.ds(0, b)], si_v.at[sl], semi[sl]).wait()
            pltpu.make_async_copy(dst_hbm.at[pl.ds(0, b)], di_v.at[sl], semi[sl]).wait()

        def gather_issue(sl):
            pltpu.async_copy(h_hbm.at[si_v.at[sl]], a_v.at[sl], semg[sl])
            pltpu.async_copy(h_hbm.at[di_v.at[sl]], b_v.at[sl], semg[sl])

        def gather_wait(sl):
            pltpu.make_async_copy(h_hbm.at[si_v.at[sl]], a_v.at[sl], semg[sl]).wait()
            pltpu.make_async_copy(h_hbm.at[di_v.at[sl]], b_v.at[sl], semg[sl]).wait()

        def compute(sl):
            @pl.loop(0, b)
            def _row(j):
                for kk in range(nv):
                    s_ = pl.ds(kk * 16, 16)
                    a_v[sl, j, s_] = jnp.abs(a_v[sl, j, s_] - b_v[sl, j, s_])

        def out_issue(j, sl):
            pltpu.async_copy(a_v.at[sl], out_hbm.at[pl.ds(base0 + j * b, b)], semo[sl])

        def out_wait(sl):
            pltpu.make_async_copy(a_v.at[sl], out_hbm.at[pl.ds(0, b)], semo[sl]).wait()

        pltpu.sync_copy(src_hbm.at[pl.ds(base0, b)], si_v.at[0])
        pltpu.sync_copy(dst_hbm.at[pl.ds(base0, b)], di_v.at[0])
        gather_issue(0)
        idx_issue(1, 1)

        @pl.loop(0, chunks - 1, step=2)
        def _main(g):
            for bb in range(2):
                j = g + bb
                osl = 1 - bb
                idx_wait(osl)

                @pl.when(j >= 1)
                def _():
                    out_wait(osl)

                gather_issue(osl)

                @pl.when(j + 2 < chunks)
                def _():
                    idx_issue(j + 2, bb)

                gather_wait(bb)
                compute(bb)
                out_issue(j, bb)

        out_wait(1)
        gather_wait(0)
        compute(0)
        pltpu.sync_copy(a_v.at[0], out_hbm.at[pl.ds(base0 + (chunks - 1) * b, b)])

    return k(h, src, dst)


def _stage1(edge_h, a1_w):
    e, d = edge_h.shape
    f = a1_w.shape[1]
    be = 2560
    g = e // be

    def body(eh_ref, w_ref, x1_ref, st_ref):
        x1 = jnp.dot(eh_ref[...], w_ref[...], preferred_element_type=jnp.float32)
        x1_ref[...] = x1

        @pl.when(pl.program_id(0) == 0)
        def _():
            st_ref[...] = jnp.zeros_like(st_ref)

        st_ref[...] += jnp.stack(
            [jnp.sum(x1, axis=0), jnp.sum(x1 * x1, axis=0)])

    return pl.pallas_call(
        body,
        grid=(g,),
        in_specs=[
            pl.BlockSpec((be, d), lambda i: (i, 0)),
            pl.BlockSpec((d, f), lambda i: (0, 0)),
        ],
        out_specs=[
            pl.BlockSpec((be, f), lambda i: (i, 0)),
            pl.BlockSpec((2, f), lambda i: (0, 0)),
        ],
        out_shape=[
            jax.ShapeDtypeStruct((e, f), jnp.float32),
            jax.ShapeDtypeStruct((2, f), jnp.float32),
        ],
    )(edge_h, a1_w)


def _stage2(x1, scale1, shift1, a2_w):
    e, f1 = x1.shape
    f2 = a2_w.shape[1]
    be = 2560
    g = e // be

    def body(x1_ref, sc_ref, sh_ref, w_ref, x2_ref, st_ref):
        y = _lrelu(x1_ref[...] * sc_ref[...] + sh_ref[...])
        x2 = jnp.dot(y, w_ref[...], preferred_element_type=jnp.float32)
        x2_ref[...] = x2

        @pl.when(pl.program_id(0) == 0)
        def _():
            st_ref[...] = jnp.zeros_like(st_ref)

        st_ref[...] += jnp.stack(
            [jnp.sum(x2, axis=0), jnp.sum(x2 * x2, axis=0)])

    return pl.pallas_call(
        body,
        grid=(g,),
        in_specs=[
            pl.BlockSpec((be, f1), lambda i: (i, 0)),
            pl.BlockSpec((1, f1), lambda i: (0, 0)),
            pl.BlockSpec((1, f1), lambda i: (0, 0)),
            pl.BlockSpec((f1, f2), lambda i: (0, 0)),
        ],
        out_specs=[
            pl.BlockSpec((be, f2), lambda i: (i, 0)),
            pl.BlockSpec((2, f2), lambda i: (0, 0)),
        ],
        out_shape=[
            jax.ShapeDtypeStruct((e, f2), jnp.float32),
            jax.ShapeDtypeStruct((2, f2), jnp.float32),
        ],
    )(x1, scale1, shift1, a2_w)


def _stage3(x2, scale2, shift2, a3_w_row, a3_b, srcc, dstc):
    e, f2 = x2.shape
    be = 2560
    g = e // be

    def body(x2_ref, sc_ref, sh_ref, w3_ref, b3_ref, s_ref, d_ref, o_ref):
        y = _lrelu(x2_ref[...] * sc_ref[...] + sh_ref[...])
        t = jnp.sum(y * w3_ref[...], axis=1, keepdims=True) + b3_ref[...]
        t = _lrelu(t)
        ee = jnp.exp(-t) + (s_ref[...] == d_ref[...]).astype(jnp.float32)
        o_ref[...] = ee

    return pl.pallas_call(
        body,
        grid=(g,),
        in_specs=[
            pl.BlockSpec((be, f2), lambda i: (i, 0)),
            pl.BlockSpec((1, f2), lambda i: (0, 0)),
            pl.BlockSpec((1, f2), lambda i: (0, 0)),
            pl.BlockSpec((1, f2), lambda i: (0, 0)),
            pl.BlockSpec((1, 1), lambda i: (0, 0)),
            pl.BlockSpec((be, 1), lambda i: (i, 0)),
            pl.BlockSpec((be, 1), lambda i: (i, 0)),
        ],
        out_specs=pl.BlockSpec((be, 1), lambda i: (i, 0)),
        out_shape=jax.ShapeDtypeStruct((e, 1), jnp.float32),
    )(x2, scale2, shift2, a3_w_row, a3_b, srcc, dstc)


def _sc_scatter(h0, h1, n, src, dst, edge_e):
    hf = h0.shape[1]
    e = src.shape[0]
    tpe = e // NS
    b = 80
    chunks = tpe // b
    nvh = hf // 16
    n_pad = ((n + 1023) // 1024) * 1024
    a_rows = n_pad // 2
    rows_per_tile = a_rows // NS
    mesh = plsc.VectorSubcoreMesh(core_axis_name="c", subcore_axis_name="s")

    @functools.partial(
        pl.kernel,
        out_type=jax.ShapeDtypeStruct((NC * a_rows, 2 * hf), jnp.float32),
        mesh=mesh,
        compiler_params=pltpu.CompilerParams(use_tc_tiling_on_sc=False),
        scratch_types=[
            pltpu.VMEM((2, b), jnp.int32),
            pltpu.VMEM((2, b), jnp.int32),
            pltpu.VMEM((2, b), jnp.int32),
            pltpu.VMEM((2, b), jnp.float32),
            pltpu.VMEM((2, b, hf), jnp.float32),
            pltpu.VMEM((2, b, 2 * hf), jnp.float32),
            pltpu.VMEM((rows_per_tile, 2 * hf), jnp.float32),
            pltpu.VMEM_SHARED((a_rows, 2 * hf), jnp.float32),
            [pltpu.SemaphoreType.DMA] * 2,
            [pltpu.SemaphoreType.DMA] * 2,
            [pltpu.SemaphoreType.DMA] * 2,
        ],
    )
    def k(h0_hbm, h1_hbm, src_hbm, dst_hbm, e_hbm, out_hbm,
          si_v, si2_v, di_v, ev_v, rows_v, sc_v, bounce_v, acc_sh,
          semi, semg, sems):
        c = lax.axis_index("c")
        s = lax.axis_index("s")
        base0 = s * tpe

        def idx_issue(j, sl):
            pltpu.async_copy(src_hbm.at[pl.ds(base0 + j * b, b)], si_v.at[sl], semi[sl])
            pltpu.async_copy(dst_hbm.at[pl.ds(base0 + j * b, b)], di_v.at[sl], semi[sl])
            pltpu.async_copy(e_hbm.at[pl.ds(base0 + j * b, b)], ev_v.at[sl], semi[sl])

        def idx_wait(sl):
            pltpu.make_async_copy(src_hbm.at[pl.ds(0, b)], si_v.at[sl], semi[sl]).wait()
            pltpu.make_async_copy(dst_hbm.at[pl.ds(0, b)], di_v.at[sl], semi[sl]).wait()
            pltpu.make_async_copy(e_hbm.at[pl.ds(0, b)], ev_v.at[sl], semi[sl]).wait()

        def gather_issue(sl):
            @pl.when(c == 0)
            def _g0():
                pltpu.async_copy(h0_hbm.at[di_v.at[sl]], rows_v.at[sl], semg[sl])

            @pl.when(c == 1)
            def _g1():
                pltpu.async_copy(h1_hbm.at[di_v.at[sl]], rows_v.at[sl], semg[sl])

        def gather_wait(sl):
            pltpu.make_async_copy(h0_hbm.at[di_v.at[sl]], rows_v.at[sl], semg[sl]).wait()

        def compute(sl):
            @pl.loop(0, b // 16)
            def _grp(gidx):
                evec = ev_v[sl, pl.ds(gidx * 16, 16)]
                sivec = si_v[sl, pl.ds(gidx * 16, 16)]
                si2_v[sl, pl.ds(gidx * 16, 16)] = lax.shift_right_logical(sivec, 1)
                e0vec = jnp.where((sivec & 1) == 0, evec, 0.0)
                e1vec = evec - e0vec
                for jj in range(16):
                    j = gidx * 16 + jj
                    e0 = e0vec[jj]
                    e1 = e1vec[jj]
                    for kk in range(nvh):
                        v = rows_v[sl, j, pl.ds(kk * 16, 16)]
                        sc_v[sl, j, pl.ds(kk * 16, 16)] = v * e0
                        sc_v[sl, j, pl.ds(hf + kk * 16, 16)] = v * e1

        def scatter_issue(sl):
            pltpu.async_copy(sc_v.at[sl], acc_sh.at[si2_v.at[sl]], sems[sl], add=True)

        def scatter_wait(sl):
            pltpu.make_async_copy(sc_v.at[sl], acc_sh.at[si2_v.at[sl]], sems[sl]).wait()

        @pl.loop(0, rows_per_tile)
        def _z(j):
            for kk in range(2 * nvh):
                bounce_v[j, pl.ds(kk * 16, 16)] = jnp.zeros((16,), jnp.float32)

        pltpu.sync_copy(bounce_v, acc_sh.at[pl.ds(s * rows_per_tile, rows_per_tile)])
        plsc.subcore_barrier()

        pltpu.sync_copy(src_hbm.at[pl.ds(base0, b)], si_v.at[0])
        pltpu.sync_copy(dst_hbm.at[pl.ds(base0, b)], di_v.at[0])
        pltpu.sync_copy(e_hbm.at[pl.ds(base0, b)], ev_v.at[0])
        gather_issue(0)
        idx_issue(1, 1)

        @pl.loop(0, chunks, step=2)
        def _main(g):
            for bb in range(2):
                j = g + bb
                osl = 1 - bb

                @pl.when(j + 1 < chunks)
                def _():
                    idx_wait(osl)
                    gather_issue(osl)

                gather_wait(bb)

                @pl.when(j >= 2)
                def _():
                    scatter_wait(bb)

                compute(bb)

                @pl.when(j + 2 < chunks)
                def _():
                    idx_issue(j + 2, bb)

                scatter_issue(bb)

        scatter_wait(0)
        scatter_wait(1)
        plsc.subcore_barrier()
        pltpu.sync_copy(acc_sh.at[pl.ds(s * rows_per_tile, rows_per_tile)], bounce_v)
        pltpu.sync_copy(bounce_v,
                        out_hbm.at[pl.ds(c * a_rows + s * rows_per_tile,
                                         rows_per_tile)])

    return k(h0, h1, src, dst, edge_e), n_pad


def _sc_rowsum(n, src, edge_e):
    e = src.shape[0]
    tpe = e // NW
    b = 80
    chunks = tpe // b
    n_pad = ((n + 1023) // 1024) * 1024
    rs_rows = n_pad // 8
    rs_per_tile = rs_rows // NS
    mesh = plsc.VectorSubcoreMesh(core_axis_name="c", subcore_axis_name="s")

    @functools.partial(
        pl.kernel,
        out_type=jax.ShapeDtypeStruct((NC * rs_rows, 128), jnp.float32),
        mesh=mesh,
        scratch_types=[
            pltpu.VMEM((b,), jnp.int32),
            pltpu.VMEM((b,), jnp.int32),
            pltpu.VMEM((b,), jnp.float32),
            pltpu.VMEM((b, 128), jnp.float32),
            pltpu.VMEM((rs_per_tile, 128), jnp.float32),
            pltpu.VMEM_SHARED((rs_rows, 128), jnp.float32),
        ],
    )
    def k(src_hbm, e_hbm, ors_hbm, si_v, si8_v, ev_v, rs_src_v, bounce_v, rs_sh):
        c = lax.axis_index("c")
        s = lax.axis_index("s")
        wid = s * NC + c
        base0 = wid * tpe
        lane = lax.iota(jnp.int32, 16)

        @pl.loop(0, b)
        def _z2(j):
            for kk in range(8):
                rs_src_v[j, pl.ds(kk * 16, 16)] = jnp.zeros((16,), jnp.float32)

        @pl.loop(0, rs_per_tile)
        def _z3(j):
            for kk in range(8):
                bounce_v[j, pl.ds(kk * 16, 16)] = jnp.zeros((16,), jnp.float32)

        pltpu.sync_copy(bounce_v, rs_sh.at[pl.ds(s * rs_per_tile, rs_per_tile)])
        plsc.subcore_barrier()

        @pl.loop(0, chunks)
        def _chunk(i):
            base = base0 + i * b
            pltpu.sync_copy(src_hbm.at[pl.ds(base, b)], si_v)
            pltpu.sync_copy(e_hbm.at[pl.ds(base, b)], ev_v)

            @pl.loop(0, b // 16)
            def _grp(gidx):
                evec = ev_v[pl.ds(gidx * 16, 16)]
                sivec = si_v[pl.ds(gidx * 16, 16)]
                si8_v[pl.ds(gidx * 16, 16)] = lax.shift_right_logical(sivec, 3)
                for jj in range(16):
                    j = gidx * 16 + jj
                    r = sivec[jj] & 7
                    rs_src_v[j, pl.ds(0, 16)] = jnp.where(lane == r, evec[jj], 0.0)

            pltpu.sync_copy(rs_src_v, rs_sh.at[si8_v], add=True)

        plsc.subcore_barrier()
        pltpu.sync_copy(rs_sh.at[pl.ds(s * rs_per_tile, rs_per_tile)], bounce_v)
        pltpu.sync_copy(bounce_v,
                        ors_hbm.at[pl.ds(c * rs_rows + s * rs_per_tile,
                                         rs_per_tile)])

    return k(src, edge_e), rs_rows


def _finalize(a0, a1, rs_col, n, d):
    bn = 2000
    g = n // bn
    hf = d // 2

    def body(a0_ref, a1_ref, r_ref, o_ref):
        rs = r_ref[...]
        rs = jnp.where(rs == 0.0, 1.0, rs)
        o_ref[...] = _lrelu(
            jnp.concatenate([a0_ref[...], a1_ref[...]], axis=1) / rs)

    return pl.pallas_call(
        body,
        grid=(g,),
        in_specs=[
            pl.BlockSpec((bn, hf), lambda i: (i, 0)),
            pl.BlockSpec((bn, hf), lambda i: (i, 0)),
            pl.BlockSpec((bn, 1), lambda i: (i, 0)),
        ],
        out_specs=pl.BlockSpec((bn, d), lambda i: (i, 0)),
        out_shape=jax.ShapeDtypeStruct((n, d), jnp.float32),
    )(a0, a1, rs_col)


def _bn_affine(stats, gamma, beta, count, eps=1e-5):
    mean = stats[0] / count
    var = stats[1] / count - mean * mean
    inv = gamma / jnp.sqrt(var + eps)
    scale = inv
    shift = beta - mean * inv
    return scale.reshape(1, -1), shift.reshape(1, -1)


def kernel(inputs, edge_index, w, a1_w, a1_b, bn1_g, bn1_b,
           a2_w, a2_b, bn2_g, bn2_b, a3_w, a3_b):
    e = edge_index.shape[1]
    src = edge_index[0].astype(jnp.int32)
    dst = edge_index[1].astype(jnp.int32)

    h, h0, h1 = _matmul_h(inputs, w)
    edge_h = _sc_gather_absdiff(h, src, dst)

    x1, st1 = _stage1(edge_h, a1_w)
    scale1, shift1 = _bn_affine(st1, bn1_g, bn1_b, e)
    x2, st2 = _stage2(x1, scale1, shift1, a2_w)
    scale2, shift2 = _bn_affine(st2, bn2_g, bn2_b, e)
    edge_e = _stage3(x2, scale2, shift2, a3_w.reshape(1, -1),
                     a3_b.reshape(1, 1), src.reshape(e, 1), dst.reshape(e, 1))

    n, d = inputs.shape[0], w.shape[1]
    ef = edge_e.reshape(e)
    acc, n_pad = _sc_scatter(h0, h1, n, src, dst, ef)
    rs, rs_rows = _sc_rowsum(n, src, ef)
    a_rows = n_pad // 2
    a0 = acc[:a_rows].reshape(n_pad, d // 2)[:n]
    a1 = acc[a_rows:].reshape(n_pad, d // 2)[:n]
    rs_col = (rs[:rs_rows, :8] + rs[rs_rows:, :8]).reshape(n_pad)[:n].reshape(n, 1)
    return _finalize(a0, a1, rs_col, n, d)


# speedup vs baseline: 1.9930x; 1.0879x over previous
import functools

import jax
import jax.numpy as jnp
from jax import lax
from jax.experimental import pallas as pl
from jax.experimental.pallas import tpu as pltpu
from jax.experimental.pallas import tpu_sc as plsc

NC = 2
NS = 16
NW = NC * NS
LRELU_SLOPE = 0.2
ACC_W = 144


def _lrelu(x):
    return jnp.where(x > 0, x, LRELU_SLOPE * x)


def _matmul_h(inputs, w):
    n, d_in = inputs.shape
    d_out = w.shape[1]
    hf = d_out // 2

    def body(x_ref, w_ref, o_ref, o0_ref, o1_ref):
        o = jnp.dot(x_ref[...], w_ref[...], preferred_element_type=jnp.float32)
        o_ref[...] = o
        o0_ref[...] = o[:, :hf]
        o1_ref[...] = o[:, hf:]

    return pl.pallas_call(
        body,
        out_shape=[
            jax.ShapeDtypeStruct((n, d_out), jnp.float32),
            jax.ShapeDtypeStruct((n, hf), jnp.float32),
            jax.ShapeDtypeStruct((n, hf), jnp.float32),
        ],
    )(inputs, w)


def _sc_gather_absdiff(h, src, dst):
    n, d = h.shape
    e = src.shape[0]
    tpe = e // NW
    b = 80
    chunks = tpe // b
    nv = d // 16
    mesh = plsc.VectorSubcoreMesh(core_axis_name="c", subcore_axis_name="s")

    @functools.partial(
        pl.kernel,
        out_type=jax.ShapeDtypeStruct((e, d), jnp.float32),
        mesh=mesh,
        scratch_types=[
            pltpu.VMEM((2, b), jnp.int32),
            pltpu.VMEM((2, b), jnp.int32),
            pltpu.VMEM((2, b, d), jnp.float32),
            pltpu.VMEM((2, b, d), jnp.float32),
            [pltpu.SemaphoreType.DMA] * 2,
            [pltpu.SemaphoreType.DMA] * 2,
            [pltpu.SemaphoreType.DMA] * 2,
        ],
    )
    def k(h_hbm, src_hbm, dst_hbm, out_hbm, si_v, di_v, a_v, b_v,
          semi, semg, semo):
        wid = lax.axis_index("s") * NC + lax.axis_index("c")
        base0 = wid * tpe

        def idx_issue(j, sl):
            pltpu.async_copy(src_hbm.at[pl.ds(base0 + j * b, b)], si_v.at[sl], semi[sl])
            pltpu.async_copy(dst_hbm.at[pl.ds(base0 + j * b, b)], di_v.at[sl], semi[sl])

        def idx_wait(sl):
            pltpu.make_async_copy(src_hbm.at[pl.ds(0, b)], si_v.at[sl], semi[sl]).wait()
            pltpu.make_async_copy(dst_hbm.at[pl.ds(0, b)], di_v.at[sl], semi[sl]).wait()

        def gather_issue(sl):
            pltpu.async_copy(h_hbm.at[si_v.at[sl]], a_v.at[sl], semg[sl])
            pltpu.async_copy(h_hbm.at[di_v.at[sl]], b_v.at[sl], semg[sl])

        def gather_wait(sl):
            pltpu.make_async_copy(h_hbm.at[si_v.at[sl]], a_v.at[sl], semg[sl]).wait()
            pltpu.make_async_copy(h_hbm.at[di_v.at[sl]], b_v.at[sl], semg[sl]).wait()

        def compute(sl):
            @pl.loop(0, b)
            def _row(j):
                for kk in range(nv):
                    s_ = pl.ds(kk * 16, 16)
                    a_v[sl, j, s_] = jnp.abs(a_v[sl, j, s_] - b_v[sl, j, s_])

        def out_issue(j, sl):
            pltpu.async_copy(a_v.at[sl], out_hbm.at[pl.ds(base0 + j * b, b)], semo[sl])

        def out_wait(sl):
            pltpu.make_async_copy(a_v.at[sl], out_hbm.at[pl.ds(0, b)], semo[sl]).wait()

        pltpu.sync_copy(src_hbm.at[pl.ds(base0, b)], si_v.at[0])
        pltpu.sync_copy(dst_hbm.at[pl.ds(base0, b)], di_v.at[0])
        gather_issue(0)
        idx_issue(1, 1)

        @pl.loop(0, chunks - 1, step=2)
        def _main(g):
            for bb in range(2):
                j = g + bb
                osl = 1 - bb
                idx_wait(osl)

                @pl.when(j >= 1)
                def _():
                    out_wait(osl)

                gather_issue(osl)

                @pl.when(j + 2 < chunks)
                def _():
                    idx_issue(j + 2, bb)

                gather_wait(bb)
                compute(bb)
                out_issue(j, bb)

        out_wait(1)
        gather_wait(0)
        compute(0)
        pltpu.sync_copy(a_v.at[0], out_hbm.at[pl.ds(base0 + (chunks - 1) * b, b)])

    return k(h, src, dst)


def _stage1(edge_h, a1_w):
    e, d = edge_h.shape
    f = a1_w.shape[1]
    be = 8000
    g = e // be

    def body(eh_ref, w_ref, x1_ref, st_ref):
        x1 = jnp.dot(eh_ref[...], w_ref[...], preferred_element_type=jnp.float32)
        x1_ref[...] = x1

        @pl.when(pl.program_id(0) == 0)
        def _():
            st_ref[...] = jnp.zeros_like(st_ref)

        st_ref[...] += jnp.stack(
            [jnp.sum(x1, axis=0), jnp.sum(x1 * x1, axis=0)])

    return pl.pallas_call(
        body,
        grid=(g,),
        in_specs=[
            pl.BlockSpec((be, d), lambda i: (i, 0)),
            pl.BlockSpec((d, f), lambda i: (0, 0)),
        ],
        out_specs=[
            pl.BlockSpec((be, f), lambda i: (i, 0)),
            pl.BlockSpec((2, f), lambda i: (0, 0)),
        ],
        out_shape=[
            jax.ShapeDtypeStruct((e, f), jnp.float32),
            jax.ShapeDtypeStruct((2, f), jnp.float32),
        ],
    )(edge_h, a1_w)


def _stage2(x1, scale1, shift1, a2_w):
    e, f1 = x1.shape
    f2 = a2_w.shape[1]
    be = 8000
    g = e // be

    def body(x1_ref, sc_ref, sh_ref, w_ref, x2_ref, st_ref):
        y = _lrelu(x1_ref[...] * sc_ref[...] + sh_ref[...])
        x2 = jnp.dot(y, w_ref[...], preferred_element_type=jnp.float32)
        x2_ref[...] = x2

        @pl.when(pl.program_id(0) == 0)
        def _():
            st_ref[...] = jnp.zeros_like(st_ref)

        st_ref[...] += jnp.stack(
            [jnp.sum(x2, axis=0), jnp.sum(x2 * x2, axis=0)])

    return pl.pallas_call(
        body,
        grid=(g,),
        in_specs=[
            pl.BlockSpec((be, f1), lambda i: (i, 0)),
            pl.BlockSpec((1, f1), lambda i: (0, 0)),
            pl.BlockSpec((1, f1), lambda i: (0, 0)),
            pl.BlockSpec((f1, f2), lambda i: (0, 0)),
        ],
        out_specs=[
            pl.BlockSpec((be, f2), lambda i: (i, 0)),
            pl.BlockSpec((2, f2), lambda i: (0, 0)),
        ],
        out_shape=[
            jax.ShapeDtypeStruct((e, f2), jnp.float32),
            jax.ShapeDtypeStruct((2, f2), jnp.float32),
        ],
    )(x1, scale1, shift1, a2_w)


def _stage3(x2, scale2, shift2, a3_w_row, a3_b, srcc, dstc):
    e, f2 = x2.shape
    be = 8000
    g = e // be

    def body(x2_ref, sc_ref, sh_ref, w3_ref, b3_ref, s_ref, d_ref, o_ref):
        y = _lrelu(x2_ref[...] * sc_ref[...] + sh_ref[...])
        t = jnp.sum(y * w3_ref[...], axis=1, keepdims=True) + b3_ref[...]
        t = _lrelu(t)
        ee = jnp.exp(-t) + (s_ref[...] == d_ref[...]).astype(jnp.float32)
        o_ref[...] = ee

    return pl.pallas_call(
        body,
        grid=(g,),
        in_specs=[
            pl.BlockSpec((be, f2), lambda i: (i, 0)),
            pl.BlockSpec((1, f2), lambda i: (0, 0)),
            pl.BlockSpec((1, f2), lambda i: (0, 0)),
            pl.BlockSpec((1, f2), lambda i: (0, 0)),
            pl.BlockSpec((1, 1), lambda i: (0, 0)),
            pl.BlockSpec((be, 1), lambda i: (i, 0)),
            pl.BlockSpec((be, 1), lambda i: (i, 0)),
        ],
        out_specs=pl.BlockSpec((be, 1), lambda i: (i, 0)),
        out_shape=jax.ShapeDtypeStruct((e, 1), jnp.float32),
    )(x2, scale2, shift2, a3_w_row, a3_b, srcc, dstc)


def _sc_scatter(h0, h1, n, src, dst, edge_e):
    hf = h0.shape[1]
    e = src.shape[0]
    tpe = e // NS
    b = 80
    chunks = tpe // b
    nvh = hf // 16
    n_pad = ((n + 1023) // 1024) * 1024
    a_rows = n_pad // 2
    rows_per_tile = a_rows // NS
    mesh = plsc.VectorSubcoreMesh(core_axis_name="c", subcore_axis_name="s")

    @functools.partial(
        pl.kernel,
        out_type=jax.ShapeDtypeStruct((NC * a_rows, 2 * hf), jnp.float32),
        mesh=mesh,
        compiler_params=pltpu.CompilerParams(use_tc_tiling_on_sc=False),
        scratch_types=[
            pltpu.VMEM((2, b), jnp.int32),
            pltpu.VMEM((2, b), jnp.int32),
            pltpu.VMEM((2, b), jnp.int32),
            pltpu.VMEM((2, b), jnp.float32),
            pltpu.VMEM((2, b, hf), jnp.float32),
            pltpu.VMEM((2, b, 2 * hf), jnp.float32),
            pltpu.VMEM((rows_per_tile, 2 * hf), jnp.float32),
            pltpu.VMEM_SHARED((a_rows, 2 * hf), jnp.float32),
            [pltpu.SemaphoreType.DMA] * 2,
            [pltpu.SemaphoreType.DMA] * 2,
            [pltpu.SemaphoreType.DMA] * 2,
        ],
    )
    def k(h0_hbm, h1_hbm, src_hbm, dst_hbm, e_hbm, out_hbm,
          si_v, si2_v, di_v, ev_v, rows_v, sc_v, bounce_v, acc_sh,
          semi, semg, sems):
        c = lax.axis_index("c")
        s = lax.axis_index("s")
        base0 = s * tpe

        def idx_issue(j, sl):
            pltpu.async_copy(src_hbm.at[pl.ds(base0 + j * b, b)], si_v.at[sl], semi[sl])
            pltpu.async_copy(dst_hbm.at[pl.ds(base0 + j * b, b)], di_v.at[sl], semi[sl])
            pltpu.async_copy(e_hbm.at[pl.ds(base0 + j * b, b)], ev_v.at[sl], semi[sl])

        def idx_wait(sl):
            pltpu.make_async_copy(src_hbm.at[pl.ds(0, b)], si_v.at[sl], semi[sl]).wait()
            pltpu.make_async_copy(dst_hbm.at[pl.ds(0, b)], di_v.at[sl], semi[sl]).wait()
            pltpu.make_async_copy(e_hbm.at[pl.ds(0, b)], ev_v.at[sl], semi[sl]).wait()

        def gather_issue(sl):
            @pl.when(c == 0)
            def _g0():
                pltpu.async_copy(h0_hbm.at[di_v.at[sl]], rows_v.at[sl], semg[sl])

            @pl.when(c == 1)
            def _g1():
                pltpu.async_copy(h1_hbm.at[di_v.at[sl]], rows_v.at[sl], semg[sl])

        def gather_wait(sl):
            pltpu.make_async_copy(h0_hbm.at[di_v.at[sl]], rows_v.at[sl], semg[sl]).wait()

        def compute(sl):
            @pl.loop(0, b // 16)
            def _grp(gidx):
                evec = ev_v[sl, pl.ds(gidx * 16, 16)]
                sivec = si_v[sl, pl.ds(gidx * 16, 16)]
                si2_v[sl, pl.ds(gidx * 16, 16)] = lax.shift_right_logical(sivec, 1)
                e0vec = jnp.where((sivec & 1) == 0, evec, 0.0)
                e1vec = evec - e0vec
                for jj in range(16):
                    j = gidx * 16 + jj
                    e0 = e0vec[jj]
                    e1 = e1vec[jj]
                    for kk in range(nvh):
                        v = rows_v[sl, j, pl.ds(kk * 16, 16)]
                        sc_v[sl, j, pl.ds(kk * 16, 16)] = v * e0
                        sc_v[sl, j, pl.ds(hf + kk * 16, 16)] = v * e1

        def scatter_issue(sl):
            pltpu.async_copy(sc_v.at[sl], acc_sh.at[si2_v.at[sl]], sems[sl], add=True)

        def scatter_wait(sl):
            pltpu.make_async_copy(sc_v.at[sl], acc_sh.at[si2_v.at[sl]], sems[sl]).wait()

        @pl.loop(0, rows_per_tile)
        def _z(j):
            for kk in range(2 * nvh):
                bounce_v[j, pl.ds(kk * 16, 16)] = jnp.zeros((16,), jnp.float32)

        pltpu.sync_copy(bounce_v, acc_sh.at[pl.ds(s * rows_per_tile, rows_per_tile)])
        plsc.subcore_barrier()

        pltpu.sync_copy(src_hbm.at[pl.ds(base0, b)], si_v.at[0])
        pltpu.sync_copy(dst_hbm.at[pl.ds(base0, b)], di_v.at[0])
        pltpu.sync_copy(e_hbm.at[pl.ds(base0, b)], ev_v.at[0])
        gather_issue(0)
        idx_issue(1, 1)

        @pl.loop(0, chunks, step=2)
        def _main(g):
            for bb in range(2):
                j = g + bb
                osl = 1 - bb

                @pl.when(j + 1 < chunks)
                def _():
                    idx_wait(osl)
                    gather_issue(osl)

                gather_wait(bb)

                @pl.when(j >= 2)
                def _():
                    scatter_wait(bb)

                compute(bb)

                @pl.when(j + 2 < chunks)
                def _():
                    idx_issue(j + 2, bb)

                scatter_issue(bb)

        scatter_wait(0)
        scatter_wait(1)
        plsc.subcore_barrier()
        pltpu.sync_copy(acc_sh.at[pl.ds(s * rows_per_tile, rows_per_tile)], bounce_v)
        pltpu.sync_copy(bounce_v,
                        out_hbm.at[pl.ds(c * a_rows + s * rows_per_tile,
                                         rows_per_tile)])

    return k(h0, h1, src, dst, edge_e), n_pad


def _sc_rowsum(n, src, edge_e):
    e = src.shape[0]
    tpe = e // NW
    b = 80
    chunks = tpe // b
    n_pad = ((n + 1023) // 1024) * 1024
    rs_rows = n_pad // 8
    rs_per_tile = rs_rows // NS
    mesh = plsc.VectorSubcoreMesh(core_axis_name="c", subcore_axis_name="s")

    @functools.partial(
        pl.kernel,
        out_type=jax.ShapeDtypeStruct((NC * rs_rows, 128), jnp.float32),
        mesh=mesh,
        scratch_types=[
            pltpu.VMEM((b,), jnp.int32),
            pltpu.VMEM((b,), jnp.int32),
            pltpu.VMEM((b,), jnp.float32),
            pltpu.VMEM((b, 128), jnp.float32),
            pltpu.VMEM((rs_per_tile, 128), jnp.float32),
            pltpu.VMEM_SHARED((rs_rows, 128), jnp.float32),
        ],
    )
    def k(src_hbm, e_hbm, ors_hbm, si_v, si8_v, ev_v, rs_src_v, bounce_v, rs_sh):
        c = lax.axis_index("c")
        s = lax.axis_index("s")
        wid = s * NC + c
        base0 = wid * tpe
        lane = lax.iota(jnp.int32, 16)

        @pl.loop(0, b)
        def _z2(j):
            for kk in range(8):
                rs_src_v[j, pl.ds(kk * 16, 16)] = jnp.zeros((16,), jnp.float32)

        @pl.loop(0, rs_per_tile)
        def _z3(j):
            for kk in range(8):
                bounce_v[j, pl.ds(kk * 16, 16)] = jnp.zeros((16,), jnp.float32)

        pltpu.sync_copy(bounce_v, rs_sh.at[pl.ds(s * rs_per_tile, rs_per_tile)])
        plsc.subcore_barrier()

        @pl.loop(0, chunks)
        def _chunk(i):
            base = base0 + i * b
            pltpu.sync_copy(src_hbm.at[pl.ds(base, b)], si_v)
            pltpu.sync_copy(e_hbm.at[pl.ds(base, b)], ev_v)

            @pl.loop(0, b // 16)
            def _grp(gidx):
                evec = ev_v[pl.ds(gidx * 16, 16)]
                sivec = si_v[pl.ds(gidx * 16, 16)]
                si8_v[pl.ds(gidx * 16, 16)] = lax.shift_right_logical(sivec, 3)
                for jj in range(16):
                    j = gidx * 16 + jj
                    r = sivec[jj] & 7
                    rs_src_v[j, pl.ds(0, 16)] = jnp.where(lane == r, evec[jj], 0.0)

            pltpu.sync_copy(rs_src_v, rs_sh.at[si8_v], add=True)

        plsc.subcore_barrier()
        pltpu.sync_copy(rs_sh.at[pl.ds(s * rs_per_tile, rs_per_tile)], bounce_v)
        pltpu.sync_copy(bounce_v,
                        ors_hbm.at[pl.ds(c * rs_rows + s * rs_per_tile,
                                         rs_per_tile)])

    return k(src, edge_e), rs_rows


def _finalize(a0, a1, rs_col, n, d):
    bn = 2000
    g = n // bn
    hf = d // 2

    def body(a0_ref, a1_ref, r_ref, o_ref):
        rs = r_ref[...]
        rs = jnp.where(rs == 0.0, 1.0, rs)
        o_ref[...] = _lrelu(
            jnp.concatenate([a0_ref[...], a1_ref[...]], axis=1) / rs)

    return pl.pallas_call(
        body,
        grid=(g,),
        in_specs=[
            pl.BlockSpec((bn, hf), lambda i: (i, 0)),
            pl.BlockSpec((bn, hf), lambda i: (i, 0)),
            pl.BlockSpec((bn, 1), lambda i: (i, 0)),
        ],
        out_specs=pl.BlockSpec((bn, d), lambda i: (i, 0)),
        out_shape=jax.ShapeDtypeStruct((n, d), jnp.float32),
    )(a0, a1, rs_col)


def _bn_affine(stats, gamma, beta, count, eps=1e-5):
    mean = stats[0] / count
    var = stats[1] / count - mean * mean
    inv = gamma / jnp.sqrt(var + eps)
    scale = inv
    shift = beta - mean * inv
    return scale.reshape(1, -1), shift.reshape(1, -1)


def kernel(inputs, edge_index, w, a1_w, a1_b, bn1_g, bn1_b,
           a2_w, a2_b, bn2_g, bn2_b, a3_w, a3_b):
    e = edge_index.shape[1]
    src = edge_index[0].astype(jnp.int32)
    dst = edge_index[1].astype(jnp.int32)

    h, h0, h1 = _matmul_h(inputs, w)
    edge_h = _sc_gather_absdiff(h, src, dst)

    x1, st1 = _stage1(edge_h, a1_w)
    scale1, shift1 = _bn_affine(st1, bn1_g, bn1_b, e)
    x2, st2 = _stage2(x1, scale1, shift1, a2_w)
    scale2, shift2 = _bn_affine(st2, bn2_g, bn2_b, e)
    edge_e = _stage3(x2, scale2, shift2, a3_w.reshape(1, -1),
                     a3_b.reshape(1, 1), src.reshape(e, 1), dst.reshape(e, 1))

    n, d = inputs.shape[0], w.shape[1]
    ef = edge_e.reshape(e)
    acc, n_pad = _sc_scatter(h0, h1, n, src, dst, ef)
    rs, rs_rows = _sc_rowsum(n, src, ef)
    a_rows = n_pad // 2
    a0 = acc[:a_rows].reshape(n_pad, d // 2)[:n]
    a1 = acc[a_rows:].reshape(n_pad, d // 2)[:n]
    rs_col = (rs[:rs_rows, :8] + rs[rs_rows:, :8]).reshape(n_pad)[:n].reshape(n, 1)
    return _finalize(a0, a1, rs_col, n, d)


# speedup vs baseline: 2.2848x; 1.1464x over previous
import functools

import jax
import jax.numpy as jnp
from jax import lax
from jax.experimental import pallas as pl
from jax.experimental.pallas import tpu as pltpu
from jax.experimental.pallas import tpu_sc as plsc

NC = 2
NS = 16
NW = NC * NS
LRELU_SLOPE = 0.2
ACC_W = 144


def _lrelu(x):
    return jnp.where(x > 0, x, LRELU_SLOPE * x)


def _matmul_h(inputs, w):
    n, d_in = inputs.shape
    d_out = w.shape[1]
    hf = d_out // 2

    def body(x_ref, w_ref, o_ref, o0_ref, o1_ref):
        o = jnp.dot(x_ref[...], w_ref[...], preferred_element_type=jnp.float32)
        o_ref[...] = o
        o0_ref[...] = o[:, :hf]
        o1_ref[...] = o[:, hf:]

    return pl.pallas_call(
        body,
        out_shape=[
            jax.ShapeDtypeStruct((n, d_out), jnp.float32),
            jax.ShapeDtypeStruct((n, hf), jnp.float32),
            jax.ShapeDtypeStruct((n, hf), jnp.float32),
        ],
    )(inputs, w)


def _sc_gather_absdiff(h, src, dst):
    n, d = h.shape
    e = src.shape[0]
    tpe = e // NW
    b = 80
    chunks = tpe // b
    nv = d // 16
    mesh = plsc.VectorSubcoreMesh(core_axis_name="c", subcore_axis_name="s")

    @functools.partial(
        pl.kernel,
        out_type=jax.ShapeDtypeStruct((e, d), jnp.float32),
        mesh=mesh,
        scratch_types=[
            pltpu.VMEM((2, b), jnp.int32),
            pltpu.VMEM((2, b), jnp.int32),
            pltpu.VMEM((2, b, d), jnp.float32),
            pltpu.VMEM((2, b, d), jnp.float32),
            [pltpu.SemaphoreType.DMA] * 2,
            [pltpu.SemaphoreType.DMA] * 2,
            [pltpu.SemaphoreType.DMA] * 2,
        ],
    )
    def k(h_hbm, src_hbm, dst_hbm, out_hbm, si_v, di_v, a_v, b_v,
          semi, semg, semo):
        wid = lax.axis_index("s") * NC + lax.axis_index("c")
        base0 = wid * tpe

        def idx_issue(j, sl):
            pltpu.async_copy(src_hbm.at[pl.ds(base0 + j * b, b)], si_v.at[sl], semi[sl])
            pltpu.async_copy(dst_hbm.at[pl.ds(base0 + j * b, b)], di_v.at[sl], semi[sl])

        def idx_wait(sl):
            pltpu.make_async_copy(src_hbm.at[pl.ds(0, b)], si_v.at[sl], semi[sl]).wait()
            pltpu.make_async_copy(dst_hbm.at[pl.ds(0, b)], di_v.at[sl], semi[sl]).wait()

        def gather_issue(sl):
            pltpu.async_copy(h_hbm.at[si_v.at[sl]], a_v.at[sl], semg[sl])
            pltpu.async_copy(h_hbm.at[di_v.at[sl]], b_v.at[sl], semg[sl])

        def gather_wait(sl):
            pltpu.make_async_copy(h_hbm.at[si_v.at[sl]], a_v.at[sl], semg[sl]).wait()
            pltpu.make_async_copy(h_hbm.at[di_v.at[sl]], b_v.at[sl], semg[sl]).wait()

        def compute(sl):
            @pl.loop(0, b)
            def _row(j):
                for kk in range(nv):
                    s_ = pl.ds(kk * 16, 16)
                    a_v[sl, j, s_] = jnp.abs(a_v[sl, j, s_] - b_v[sl, j, s_])

        def out_issue(j, sl):
            pltpu.async_copy(a_v.at[sl], out_hbm.at[pl.ds(base0 + j * b, b)], semo[sl])

        def out_wait(sl):
            pltpu.make_async_copy(a_v.at[sl], out_hbm.at[pl.ds(0, b)], semo[sl]).wait()

        pltpu.sync_copy(src_hbm.at[pl.ds(base0, b)], si_v.at[0])
        pltpu.sync_copy(dst_hbm.at[pl.ds(base0, b)], di_v.at[0])
        gather_issue(0)
        idx_issue(1, 1)

        @pl.loop(0, chunks - 1, step=2)
        def _main(g):
            for bb in range(2):
                j = g + bb
                osl = 1 - bb
                idx_wait(osl)

                @pl.when(j >= 1)
                def _():
                    out_wait(osl)

                gather_issue(osl)

                @pl.when(j + 2 < chunks)
                def _():
                    idx_issue(j + 2, bb)

                gather_wait(bb)
                compute(bb)
                out_issue(j, bb)

        out_wait(1)
        gather_wait(0)
        compute(0)
        pltpu.sync_copy(a_v.at[0], out_hbm.at[pl.ds(base0 + (chunks - 1) * b, b)])

    return k(h, src, dst)


def _stage1(edge_h, a1_w):
    e, d = edge_h.shape
    f = a1_w.shape[1]
    be = 16000
    g = e // be

    def body(eh_ref, w_ref, x1_ref, st_ref):
        x1 = jnp.dot(eh_ref[...], w_ref[...], preferred_element_type=jnp.float32)
        x1_ref[...] = x1

        @pl.when(pl.program_id(0) == 0)
        def _():
            st_ref[...] = jnp.zeros_like(st_ref)

        st_ref[...] += jnp.stack(
            [jnp.sum(x1, axis=0), jnp.sum(x1 * x1, axis=0)])

    return pl.pallas_call(
        body,
        grid=(g,),
        in_specs=[
            pl.BlockSpec((be, d), lambda i: (i, 0)),
            pl.BlockSpec((d, f), lambda i: (0, 0)),
        ],
        out_specs=[
            pl.BlockSpec((be, f), lambda i: (i, 0)),
            pl.BlockSpec((2, f), lambda i: (0, 0)),
        ],
        out_shape=[
            jax.ShapeDtypeStruct((e, f), jnp.float32),
            jax.ShapeDtypeStruct((2, f), jnp.float32),
        ],
    )(edge_h, a1_w)


def _stage2(x1, scale1, shift1, a2_w):
    e, f1 = x1.shape
    f2 = a2_w.shape[1]
    be = 16000
    g = e // be
    ep = ((e + 128 * 8 * 64 - 1) // (128 * 64 * 8)) * (128 * 64 * 8)

    def body(x1_ref, sc_ref, sh_ref, w_ref, x2_ref, st_ref):
        y = _lrelu(x1_ref[...] * sc_ref[...] + sh_ref[...])
        x2 = jnp.dot(y, w_ref[...], preferred_element_type=jnp.float32)
        x2_ref[...] = x2

        @pl.when(pl.program_id(0) == 0)
        def _():
            st_ref[...] = jnp.zeros_like(st_ref)

        st_ref[...] += jnp.stack(
            [jnp.sum(x2, axis=0), jnp.sum(x2 * x2, axis=0)])

    return pl.pallas_call(
        body,
        grid=(g,),
        in_specs=[
            pl.BlockSpec((be, f1), lambda i: (i, 0)),
            pl.BlockSpec((1, f1), lambda i: (0, 0)),
            pl.BlockSpec((1, f1), lambda i: (0, 0)),
            pl.BlockSpec((f1, f2), lambda i: (0, 0)),
        ],
        out_specs=[
            pl.BlockSpec((be, f2), lambda i: (i, 0)),
            pl.BlockSpec((2, f2), lambda i: (0, 0)),
        ],
        out_shape=[
            jax.ShapeDtypeStruct((ep, f2), jnp.float32),
            jax.ShapeDtypeStruct((2, f2), jnp.float32),
        ],
    )(x1, scale1, shift1, a2_w)


def _stage3(x2p, scale2, shift2, a3_w_row, a3_b, src2, dst2):
    ep, f2 = x2p.shape
    rows = ep // 128
    br = 64
    g = rows // br
    be = br * 128

    def body(x2_ref, sc_ref, sh_ref, w3_ref, b3_ref, s_ref, d_ref, o_ref):
        y = _lrelu(x2_ref[...] * sc_ref[...] + sh_ref[...])
        t = jnp.sum(y.reshape(br, 128, f2) * w3_ref[...], axis=2) + b3_ref[0, 0]
        t = _lrelu(t)
        o_ref[...] = jnp.exp(-t) + (s_ref[...] == d_ref[...]).astype(jnp.float32)

    return pl.pallas_call(
        body,
        grid=(g,),
        in_specs=[
            pl.BlockSpec((be, f2), lambda i: (i, 0)),
            pl.BlockSpec((1, f2), lambda i: (0, 0)),
            pl.BlockSpec((1, f2), lambda i: (0, 0)),
            pl.BlockSpec((1, 1, f2), lambda i: (0, 0, 0)),
            pl.BlockSpec((1, 1), lambda i: (0, 0)),
            pl.BlockSpec((br, 128), lambda i: (i, 0)),
            pl.BlockSpec((br, 128), lambda i: (i, 0)),
        ],
        out_specs=pl.BlockSpec((br, 128), lambda i: (i, 0)),
        out_shape=jax.ShapeDtypeStruct((rows, 128), jnp.float32),
    )(x2p, scale2, shift2, a3_w_row, a3_b, src2, dst2)


def _sc_scatter(h0, h1, n, src, dst, edge_e):
    hf = h0.shape[1]
    e = src.shape[0]
    tpe = e // NS
    b = 80
    chunks = tpe // b
    nvh = hf // 16
    n_pad = ((n + 1023) // 1024) * 1024
    a_rows = n_pad // 2
    rows_per_tile = a_rows // NS
    mesh = plsc.VectorSubcoreMesh(core_axis_name="c", subcore_axis_name="s")

    @functools.partial(
        pl.kernel,
        out_type=jax.ShapeDtypeStruct((NC * a_rows, 2 * hf), jnp.float32),
        mesh=mesh,
        compiler_params=pltpu.CompilerParams(use_tc_tiling_on_sc=False),
        scratch_types=[
            pltpu.VMEM((2, b), jnp.int32),
            pltpu.VMEM((2, b), jnp.int32),
            pltpu.VMEM((2, b), jnp.int32),
            pltpu.VMEM((2, b), jnp.float32),
            pltpu.VMEM((2, b, hf), jnp.float32),
            pltpu.VMEM((2, b, 2 * hf), jnp.float32),
            pltpu.VMEM((rows_per_tile, 2 * hf), jnp.float32),
            pltpu.VMEM_SHARED((a_rows, 2 * hf), jnp.float32),
            [pltpu.SemaphoreType.DMA] * 2,
            [pltpu.SemaphoreType.DMA] * 2,
            [pltpu.SemaphoreType.DMA] * 2,
        ],
    )
    def k(h0_hbm, h1_hbm, src_hbm, dst_hbm, e_hbm, out_hbm,
          si_v, si2_v, di_v, ev_v, rows_v, sc_v, bounce_v, acc_sh,
          semi, semg, sems):
        c = lax.axis_index("c")
        s = lax.axis_index("s")
        base0 = s * tpe

        def idx_issue(j, sl):
            pltpu.async_copy(src_hbm.at[pl.ds(base0 + j * b, b)], si_v.at[sl], semi[sl])
            pltpu.async_copy(dst_hbm.at[pl.ds(base0 + j * b, b)], di_v.at[sl], semi[sl])
            pltpu.async_copy(e_hbm.at[pl.ds(base0 + j * b, b)], ev_v.at[sl], semi[sl])

        def idx_wait(sl):
            pltpu.make_async_copy(src_hbm.at[pl.ds(0, b)], si_v.at[sl], semi[sl]).wait()
            pltpu.make_async_copy(dst_hbm.at[pl.ds(0, b)], di_v.at[sl], semi[sl]).wait()
            pltpu.make_async_copy(e_hbm.at[pl.ds(0, b)], ev_v.at[sl], semi[sl]).wait()

        def gather_issue(sl):
            @pl.when(c == 0)
            def _g0():
                pltpu.async_copy(h0_hbm.at[di_v.at[sl]], rows_v.at[sl], semg[sl])

            @pl.when(c == 1)
            def _g1():
                pltpu.async_copy(h1_hbm.at[di_v.at[sl]], rows_v.at[sl], semg[sl])

        def gather_wait(sl):
            pltpu.make_async_copy(h0_hbm.at[di_v.at[sl]], rows_v.at[sl], semg[sl]).wait()

        def compute(sl):
            @pl.loop(0, b // 16)
            def _grp(gidx):
                evec = ev_v[sl, pl.ds(gidx * 16, 16)]
                sivec = si_v[sl, pl.ds(gidx * 16, 16)]
                si2_v[sl, pl.ds(gidx * 16, 16)] = lax.shift_right_logical(sivec, 1)
                e0vec = jnp.where((sivec & 1) == 0, evec, 0.0)
                e1vec = evec - e0vec
                for jj in range(16):
                    j = gidx * 16 + jj
                    e0 = e0vec[jj]
                    e1 = e1vec[jj]
                    for kk in range(nvh):
                        v = rows_v[sl, j, pl.ds(kk * 16, 16)]
                        sc_v[sl, j, pl.ds(kk * 16, 16)] = v * e0
                        sc_v[sl, j, pl.ds(hf + kk * 16, 16)] = v * e1

        def scatter_issue(sl):
            pltpu.async_copy(sc_v.at[sl], acc_sh.at[si2_v.at[sl]], sems[sl], add=True)

        def scatter_wait(sl):
            pltpu.make_async_copy(sc_v.at[sl], acc_sh.at[si2_v.at[sl]], sems[sl]).wait()

        @pl.loop(0, rows_per_tile)
        def _z(j):
            for kk in range(2 * nvh):
                bounce_v[j, pl.ds(kk * 16, 16)] = jnp.zeros((16,), jnp.float32)

        pltpu.sync_copy(bounce_v, acc_sh.at[pl.ds(s * rows_per_tile, rows_per_tile)])
        plsc.subcore_barrier()

        pltpu.sync_copy(src_hbm.at[pl.ds(base0, b)], si_v.at[0])
        pltpu.sync_copy(dst_hbm.at[pl.ds(base0, b)], di_v.at[0])
        pltpu.sync_copy(e_hbm.at[pl.ds(base0, b)], ev_v.at[0])
        gather_issue(0)
        idx_issue(1, 1)

        @pl.loop(0, chunks, step=2)
        def _main(g):
            for bb in range(2):
                j = g + bb
                osl = 1 - bb

                @pl.when(j + 1 < chunks)
                def _():
                    idx_wait(osl)
                    gather_issue(osl)

                gather_wait(bb)

                @pl.when(j >= 2)
                def _():
                    scatter_wait(bb)

                compute(bb)

                @pl.when(j + 2 < chunks)
                def _():
                    idx_issue(j + 2, bb)

                scatter_issue(bb)

        scatter_wait(0)
        scatter_wait(1)
        plsc.subcore_barrier()
        pltpu.sync_copy(acc_sh.at[pl.ds(s * rows_per_tile, rows_per_tile)], bounce_v)
        pltpu.sync_copy(bounce_v,
                        out_hbm.at[pl.ds(c * a_rows + s * rows_per_tile,
                                         rows_per_tile)])

    return k(h0, h1, src, dst, edge_e), n_pad


def _sc_rowsum(n, src, edge_e):
    e = src.shape[0]
    tpe = e // NW
    b = 80
    chunks = tpe // b
    n_pad = ((n + 1023) // 1024) * 1024
    rs_rows = n_pad // 8
    rs_per_tile = rs_rows // NS
    mesh = plsc.VectorSubcoreMesh(core_axis_name="c", subcore_axis_name="s")

    @functools.partial(
        pl.kernel,
        out_type=jax.ShapeDtypeStruct((NC * rs_rows, 128), jnp.float32),
        mesh=mesh,
        scratch_types=[
            pltpu.VMEM((b,), jnp.int32),
            pltpu.VMEM((b,), jnp.int32),
            pltpu.VMEM((b,), jnp.float32),
            pltpu.VMEM((b, 128), jnp.float32),
            pltpu.VMEM((rs_per_tile, 128), jnp.float32),
            pltpu.VMEM_SHARED((rs_rows, 128), jnp.float32),
        ],
    )
    def k(src_hbm, e_hbm, ors_hbm, si_v, si8_v, ev_v, rs_src_v, bounce_v, rs_sh):
        c = lax.axis_index("c")
        s = lax.axis_index("s")
        wid = s * NC + c
        base0 = wid * tpe
        lane = lax.iota(jnp.int32, 16)

        @pl.loop(0, b)
        def _z2(j):
            for kk in range(8):
                rs_src_v[j, pl.ds(kk * 16, 16)] = jnp.zeros((16,), jnp.float32)

        @pl.loop(0, rs_per_tile)
        def _z3(j):
            for kk in range(8):
                bounce_v[j, pl.ds(kk * 16, 16)] = jnp.zeros((16,), jnp.float32)

        pltpu.sync_copy(bounce_v, rs_sh.at[pl.ds(s * rs_per_tile, rs_per_tile)])
        plsc.subcore_barrier()

        @pl.loop(0, chunks)
        def _chunk(i):
            base = base0 + i * b
            pltpu.sync_copy(src_hbm.at[pl.ds(base, b)], si_v)
            pltpu.sync_copy(e_hbm.at[pl.ds(base, b)], ev_v)

            @pl.loop(0, b // 16)
            def _grp(gidx):
                evec = ev_v[pl.ds(gidx * 16, 16)]
                sivec = si_v[pl.ds(gidx * 16, 16)]
                si8_v[pl.ds(gidx * 16, 16)] = lax.shift_right_logical(sivec, 3)
                for jj in range(16):
                    j = gidx * 16 + jj
                    r = sivec[jj] & 7
                    rs_src_v[j, pl.ds(0, 16)] = jnp.where(lane == r, evec[jj], 0.0)

            pltpu.sync_copy(rs_src_v, rs_sh.at[si8_v], add=True)

        plsc.subcore_barrier()
        pltpu.sync_copy(rs_sh.at[pl.ds(s * rs_per_tile, rs_per_tile)], bounce_v)
        pltpu.sync_copy(bounce_v,
                        ors_hbm.at[pl.ds(c * rs_rows + s * rs_per_tile,
                                         rs_per_tile)])

    return k(src, edge_e), rs_rows


def _finalize(a0, a1, rs_col, n, d):
    bn = 2000
    g = n // bn
    hf = d // 2

    def body(a0_ref, a1_ref, r_ref, o_ref):
        rs = r_ref[...]
        rs = jnp.where(rs == 0.0, 1.0, rs)
        o_ref[...] = _lrelu(
            jnp.concatenate([a0_ref[...], a1_ref[...]], axis=1) / rs)

    return pl.pallas_call(
        body,
        grid=(g,),
        in_specs=[
            pl.BlockSpec((bn, hf), lambda i: (i, 0)),
            pl.BlockSpec((bn, hf), lambda i: (i, 0)),
            pl.BlockSpec((bn, 1), lambda i: (i, 0)),
        ],
        out_specs=pl.BlockSpec((bn, d), lambda i: (i, 0)),
        out_shape=jax.ShapeDtypeStruct((n, d), jnp.float32),
    )(a0, a1, rs_col)


def _bn_affine(stats, gamma, beta, count, eps=1e-5):
    mean = stats[0] / count
    var = stats[1] / count - mean * mean
    inv = gamma / jnp.sqrt(var + eps)
    scale = inv
    shift = beta - mean * inv
    return scale.reshape(1, -1), shift.reshape(1, -1)


def kernel(inputs, edge_index, w, a1_w, a1_b, bn1_g, bn1_b,
           a2_w, a2_b, bn2_g, bn2_b, a3_w, a3_b):
    e = edge_index.shape[1]
    src = edge_index[0].astype(jnp.int32)
    dst = edge_index[1].astype(jnp.int32)

    h, h0, h1 = _matmul_h(inputs, w)
    edge_h = _sc_gather_absdiff(h, src, dst)

    x1, st1 = _stage1(edge_h, a1_w)
    scale1, shift1 = _bn_affine(st1, bn1_g, bn1_b, e)
    x2, st2 = _stage2(x1, scale1, shift1, a2_w)
    scale2, shift2 = _bn_affine(st2, bn2_g, bn2_b, e)
    ep = x2.shape[0]
    pad = ep - e
    src2 = jnp.concatenate([src, jnp.zeros((pad,), jnp.int32)]).reshape(ep // 128, 128)
    dst2 = jnp.concatenate([dst, jnp.ones((pad,), jnp.int32)]).reshape(ep // 128, 128)
    edge_e = _stage3(x2, scale2, shift2, a3_w.reshape(1, 1, -1),
                     a3_b.reshape(1, 1), src2, dst2)

    n, d = inputs.shape[0], w.shape[1]
    ef = edge_e.reshape(ep)[:e]
    acc, n_pad = _sc_scatter(h0, h1, n, src, dst, ef)
    rs, rs_rows = _sc_rowsum(n, src, ef)
    a_rows = n_pad // 2
    a0 = acc[:a_rows].reshape(n_pad, d // 2)[:n]
    a1 = acc[a_rows:].reshape(n_pad, d // 2)[:n]
    rs_col = (rs[:rs_rows, :8] + rs[rs_rows:, :8]).reshape(n_pad)[:n].reshape(n, 1)
    return _finalize(a0, a1, rs_col, n, d)


# speedup vs baseline: 2.9359x; 1.2850x over previous
import functools

import jax
import jax.numpy as jnp
from jax import lax
from jax.experimental import pallas as pl
from jax.experimental.pallas import tpu as pltpu
from jax.experimental.pallas import tpu_sc as plsc

NC = 2
NS = 16
NW = NC * NS
LRELU_SLOPE = 0.2
ACC_W = 144


def _lrelu(x):
    return jnp.where(x > 0, x, LRELU_SLOPE * x)


def _matmul_h(inputs, w):
    n, d_in = inputs.shape
    d_out = w.shape[1]
    hf = d_out // 2

    def body(x_ref, w_ref, o_ref, o0_ref, o1_ref):
        o = jnp.dot(x_ref[...], w_ref[...], preferred_element_type=jnp.float32)
        o_ref[...] = o
        o0_ref[...] = o[:, :hf]
        o1_ref[...] = o[:, hf:]

    return pl.pallas_call(
        body,
        out_shape=[
            jax.ShapeDtypeStruct((n, d_out), jnp.float32),
            jax.ShapeDtypeStruct((n, hf), jnp.float32),
            jax.ShapeDtypeStruct((n, hf), jnp.float32),
        ],
    )(inputs, w)


def _sc_gather_absdiff(h, src, dst):
    n, d = h.shape
    e = src.shape[0]
    tpe = e // NW
    b = 80
    chunks = tpe // b
    nv = d // 16
    mesh = plsc.VectorSubcoreMesh(core_axis_name="c", subcore_axis_name="s")

    @functools.partial(
        pl.kernel,
        out_type=jax.ShapeDtypeStruct((e, d), jnp.float32),
        mesh=mesh,
        scratch_types=[
            pltpu.VMEM((2, b), jnp.int32),
            pltpu.VMEM((2, b), jnp.int32),
            pltpu.VMEM((2, b, d), jnp.float32),
            pltpu.VMEM((2, b, d), jnp.float32),
            [pltpu.SemaphoreType.DMA] * 2,
            [pltpu.SemaphoreType.DMA] * 2,
            [pltpu.SemaphoreType.DMA] * 2,
        ],
    )
    def k(h_hbm, src_hbm, dst_hbm, out_hbm, si_v, di_v, a_v, b_v,
          semi, semg, semo):
        wid = lax.axis_index("s") * NC + lax.axis_index("c")
        base0 = wid * tpe

        def idx_issue(j, sl):
            pltpu.async_copy(src_hbm.at[pl.ds(base0 + j * b, b)], si_v.at[sl], semi[sl])
            pltpu.async_copy(dst_hbm.at[pl.ds(base0 + j * b, b)], di_v.at[sl], semi[sl])

        def idx_wait(sl):
            pltpu.make_async_copy(src_hbm.at[pl.ds(0, b)], si_v.at[sl], semi[sl]).wait()
            pltpu.make_async_copy(dst_hbm.at[pl.ds(0, b)], di_v.at[sl], semi[sl]).wait()

        def gather_issue(sl):
            pltpu.async_copy(h_hbm.at[si_v.at[sl]], a_v.at[sl], semg[sl])
            pltpu.async_copy(h_hbm.at[di_v.at[sl]], b_v.at[sl], semg[sl])

        def gather_wait(sl):
            pltpu.make_async_copy(h_hbm.at[si_v.at[sl]], a_v.at[sl], semg[sl]).wait()
            pltpu.make_async_copy(h_hbm.at[di_v.at[sl]], b_v.at[sl], semg[sl]).wait()

        def compute(sl):
            @plsc.parallel_loop(0, b, unroll=4)
            def _row(j):
                for kk in range(nv):
                    s_ = pl.ds(kk * 16, 16)
                    a_v[sl, j, s_] = jnp.abs(a_v[sl, j, s_] - b_v[sl, j, s_])

        def out_issue(j, sl):
            pltpu.async_copy(a_v.at[sl], out_hbm.at[pl.ds(base0 + j * b, b)], semo[sl])

        def out_wait(sl):
            pltpu.make_async_copy(a_v.at[sl], out_hbm.at[pl.ds(0, b)], semo[sl]).wait()

        pltpu.sync_copy(src_hbm.at[pl.ds(base0, b)], si_v.at[0])
        pltpu.sync_copy(dst_hbm.at[pl.ds(base0, b)], di_v.at[0])
        gather_issue(0)
        idx_issue(1, 1)

        @pl.loop(0, chunks - 1, step=2)
        def _main(g):
            for bb in range(2):
                j = g + bb
                osl = 1 - bb
                idx_wait(osl)

                @pl.when(j >= 1)
                def _():
                    out_wait(osl)

                gather_issue(osl)

                @pl.when(j + 2 < chunks)
                def _():
                    idx_issue(j + 2, bb)

                gather_wait(bb)
                compute(bb)
                out_issue(j, bb)

        out_wait(1)
        gather_wait(0)
        compute(0)
        pltpu.sync_copy(a_v.at[0], out_hbm.at[pl.ds(base0 + (chunks - 1) * b, b)])

    return k(h, src, dst)


def _stage1(edge_h, a1_w):
    e, d = edge_h.shape
    f = a1_w.shape[1]
    be = 16000
    g = e // be

    def body(eh_ref, w_ref, x1_ref, st_ref):
        x1 = jnp.dot(eh_ref[...], w_ref[...], preferred_element_type=jnp.float32)
        x1_ref[...] = x1

        @pl.when(pl.program_id(0) == 0)
        def _():
            st_ref[...] = jnp.zeros_like(st_ref)

        st_ref[...] += jnp.stack(
            [jnp.sum(x1, axis=0), jnp.sum(x1 * x1, axis=0)])

    return pl.pallas_call(
        body,
        grid=(g,),
        in_specs=[
            pl.BlockSpec((be, d), lambda i: (i, 0)),
            pl.BlockSpec((d, f), lambda i: (0, 0)),
        ],
        out_specs=[
            pl.BlockSpec((be, f), lambda i: (i, 0)),
            pl.BlockSpec((2, f), lambda i: (0, 0)),
        ],
        out_shape=[
            jax.ShapeDtypeStruct((e, f), jnp.float32),
            jax.ShapeDtypeStruct((2, f), jnp.float32),
        ],
    )(edge_h, a1_w)


def _stage2(x1, scale1, shift1, a2_w):
    e, f1 = x1.shape
    f2 = a2_w.shape[1]
    be = 16000
    g = e // be
    ep = ((e + 128 * 8 * 64 - 1) // (128 * 64 * 8)) * (128 * 64 * 8)

    def body(x1_ref, sc_ref, sh_ref, w_ref, x2_ref, st_ref):
        y = _lrelu(x1_ref[...] * sc_ref[...] + sh_ref[...])
        x2 = jnp.dot(y, w_ref[...], preferred_element_type=jnp.float32)
        x2_ref[...] = x2

        @pl.when(pl.program_id(0) == 0)
        def _():
            st_ref[...] = jnp.zeros_like(st_ref)

        st_ref[...] += jnp.stack(
            [jnp.sum(x2, axis=0), jnp.sum(x2 * x2, axis=0)])

    return pl.pallas_call(
        body,
        grid=(g,),
        in_specs=[
            pl.BlockSpec((be, f1), lambda i: (i, 0)),
            pl.BlockSpec((1, f1), lambda i: (0, 0)),
            pl.BlockSpec((1, f1), lambda i: (0, 0)),
            pl.BlockSpec((f1, f2), lambda i: (0, 0)),
        ],
        out_specs=[
            pl.BlockSpec((be, f2), lambda i: (i, 0)),
            pl.BlockSpec((2, f2), lambda i: (0, 0)),
        ],
        out_shape=[
            jax.ShapeDtypeStruct((ep, f2), jnp.float32),
            jax.ShapeDtypeStruct((2, f2), jnp.float32),
        ],
    )(x1, scale1, shift1, a2_w)


def _stage3(x2p, scale2, shift2, a3_w_row, a3_b, src2, dst2):
    ep, f2 = x2p.shape
    rows = ep // 128
    br = 64
    g = rows // br
    be = br * 128

    def body(x2_ref, sc_ref, sh_ref, w3_ref, b3_ref, s_ref, d_ref, o_ref):
        y = _lrelu(x2_ref[...] * sc_ref[...] + sh_ref[...])
        t = jnp.sum(y.reshape(br, 128, f2) * w3_ref[...], axis=2) + b3_ref[0, 0]
        t = _lrelu(t)
        o_ref[...] = jnp.exp(-t) + (s_ref[...] == d_ref[...]).astype(jnp.float32)

    return pl.pallas_call(
        body,
        grid=(g,),
        in_specs=[
            pl.BlockSpec((be, f2), lambda i: (i, 0)),
            pl.BlockSpec((1, f2), lambda i: (0, 0)),
            pl.BlockSpec((1, f2), lambda i: (0, 0)),
            pl.BlockSpec((1, 1, f2), lambda i: (0, 0, 0)),
            pl.BlockSpec((1, 1), lambda i: (0, 0)),
            pl.BlockSpec((br, 128), lambda i: (i, 0)),
            pl.BlockSpec((br, 128), lambda i: (i, 0)),
        ],
        out_specs=pl.BlockSpec((br, 128), lambda i: (i, 0)),
        out_shape=jax.ShapeDtypeStruct((rows, 128), jnp.float32),
    )(x2p, scale2, shift2, a3_w_row, a3_b, src2, dst2)


def _sc_scatter(h0, h1, n, src, dst, edge_e):
    hf = h0.shape[1]
    e = src.shape[0]
    tpe = e // NS
    b = 80
    chunks = tpe // b
    nvh = hf // 16
    n_pad = ((n + 1023) // 1024) * 1024
    a_rows = n_pad // 2
    rows_per_tile = a_rows // NS
    mesh = plsc.VectorSubcoreMesh(core_axis_name="c", subcore_axis_name="s")

    @functools.partial(
        pl.kernel,
        out_type=jax.ShapeDtypeStruct((NC * a_rows, 2 * hf), jnp.float32),
        mesh=mesh,
        compiler_params=pltpu.CompilerParams(use_tc_tiling_on_sc=False),
        scratch_types=[
            pltpu.VMEM((2, b), jnp.int32),
            pltpu.VMEM((2, b), jnp.int32),
            pltpu.VMEM((2, b), jnp.int32),
            pltpu.VMEM((2, b), jnp.float32),
            pltpu.VMEM((2, b, hf), jnp.float32),
            pltpu.VMEM((2, b, 2 * hf), jnp.float32),
            pltpu.VMEM((rows_per_tile, 2 * hf), jnp.float32),
            pltpu.VMEM_SHARED((a_rows, 2 * hf), jnp.float32),
            [pltpu.SemaphoreType.DMA] * 2,
            [pltpu.SemaphoreType.DMA] * 2,
            [pltpu.SemaphoreType.DMA] * 2,
        ],
    )
    def k(h0_hbm, h1_hbm, src_hbm, dst_hbm, e_hbm, out_hbm,
          si_v, si2_v, di_v, ev_v, rows_v, sc_v, bounce_v, acc_sh,
          semi, semg, sems):
        c = lax.axis_index("c")
        s = lax.axis_index("s")
        base0 = s * tpe

        def idx_issue(j, sl):
            pltpu.async_copy(src_hbm.at[pl.ds(base0 + j * b, b)], si_v.at[sl], semi[sl])
            pltpu.async_copy(dst_hbm.at[pl.ds(base0 + j * b, b)], di_v.at[sl], semi[sl])
            pltpu.async_copy(e_hbm.at[pl.ds(base0 + j * b, b)], ev_v.at[sl], semi[sl])

        def idx_wait(sl):
            pltpu.make_async_copy(src_hbm.at[pl.ds(0, b)], si_v.at[sl], semi[sl]).wait()
            pltpu.make_async_copy(dst_hbm.at[pl.ds(0, b)], di_v.at[sl], semi[sl]).wait()
            pltpu.make_async_copy(e_hbm.at[pl.ds(0, b)], ev_v.at[sl], semi[sl]).wait()

        def gather_issue(sl):
            @pl.when(c == 0)
            def _g0():
                pltpu.async_copy(h0_hbm.at[di_v.at[sl]], rows_v.at[sl], semg[sl])

            @pl.when(c == 1)
            def _g1():
                pltpu.async_copy(h1_hbm.at[di_v.at[sl]], rows_v.at[sl], semg[sl])

        def gather_wait(sl):
            pltpu.make_async_copy(h0_hbm.at[di_v.at[sl]], rows_v.at[sl], semg[sl]).wait()

        def compute(sl):
            @plsc.parallel_loop(0, b // 16, unroll=5)
            def _grp(gidx):
                evec = ev_v[sl, pl.ds(gidx * 16, 16)]
                sivec = si_v[sl, pl.ds(gidx * 16, 16)]
                si2_v[sl, pl.ds(gidx * 16, 16)] = lax.shift_right_logical(sivec, 1)
                e0vec = jnp.where((sivec & 1) == 0, evec, 0.0)
                e1vec = evec - e0vec
                for jj in range(16):
                    j = gidx * 16 + jj
                    e0 = e0vec[jj]
                    e1 = e1vec[jj]
                    for kk in range(nvh):
                        v = rows_v[sl, j, pl.ds(kk * 16, 16)]
                        sc_v[sl, j, pl.ds(kk * 16, 16)] = v * e0
                        sc_v[sl, j, pl.ds(hf + kk * 16, 16)] = v * e1

        def scatter_issue(sl):
            pltpu.async_copy(sc_v.at[sl], acc_sh.at[si2_v.at[sl]], sems[sl], add=True)

        def scatter_wait(sl):
            pltpu.make_async_copy(sc_v.at[sl], acc_sh.at[si2_v.at[sl]], sems[sl]).wait()

        @pl.loop(0, rows_per_tile)
        def _z(j):
            for kk in range(2 * nvh):
                bounce_v[j, pl.ds(kk * 16, 16)] = jnp.zeros((16,), jnp.float32)

        pltpu.sync_copy(bounce_v, acc_sh.at[pl.ds(s * rows_per_tile, rows_per_tile)])
        plsc.subcore_barrier()

        pltpu.sync_copy(src_hbm.at[pl.ds(base0, b)], si_v.at[0])
        pltpu.sync_copy(dst_hbm.at[pl.ds(base0, b)], di_v.at[0])
        pltpu.sync_copy(e_hbm.at[pl.ds(base0, b)], ev_v.at[0])
        gather_issue(0)
        idx_issue(1, 1)

        @pl.loop(0, chunks, step=2)
        def _main(g):
            for bb in range(2):
                j = g + bb
                osl = 1 - bb

                @pl.when(j + 1 < chunks)
                def _():
                    idx_wait(osl)
                    gather_issue(osl)

                gather_wait(bb)

                @pl.when(j >= 2)
                def _():
                    scatter_wait(bb)

                compute(bb)

                @pl.when(j + 2 < chunks)
                def _():
                    idx_issue(j + 2, bb)

                scatter_issue(bb)

        scatter_wait(0)
        scatter_wait(1)
        plsc.subcore_barrier()
        pltpu.sync_copy(acc_sh.at[pl.ds(s * rows_per_tile, rows_per_tile)], bounce_v)
        pltpu.sync_copy(bounce_v,
                        out_hbm.at[pl.ds(c * a_rows + s * rows_per_tile,
                                         rows_per_tile)])

    return k(h0, h1, src, dst, edge_e), n_pad


def _sc_rowsum(n, src, edge_e):
    e = src.shape[0]
    tpe = e // NW
    b = 80
    chunks = tpe // b
    n_pad = ((n + 1023) // 1024) * 1024
    rs_rows = n_pad // 8
    rs_per_tile = rs_rows // NS
    mesh = plsc.VectorSubcoreMesh(core_axis_name="c", subcore_axis_name="s")

    @functools.partial(
        pl.kernel,
        out_type=jax.ShapeDtypeStruct((NC * rs_rows, 128), jnp.float32),
        mesh=mesh,
        scratch_types=[
            pltpu.VMEM((b,), jnp.int32),
            pltpu.VMEM((b,), jnp.int32),
            pltpu.VMEM((b,), jnp.float32),
            pltpu.VMEM((b, 128), jnp.float32),
            pltpu.VMEM((rs_per_tile, 128), jnp.float32),
            pltpu.VMEM_SHARED((rs_rows, 128), jnp.float32),
        ],
    )
    def k(src_hbm, e_hbm, ors_hbm, si_v, si8_v, ev_v, rs_src_v, bounce_v, rs_sh):
        c = lax.axis_index("c")
        s = lax.axis_index("s")
        wid = s * NC + c
        base0 = wid * tpe
        lane = lax.iota(jnp.int32, 16)

        @pl.loop(0, b)
        def _z2(j):
            for kk in range(8):
                rs_src_v[j, pl.ds(kk * 16, 16)] = jnp.zeros((16,), jnp.float32)

        @pl.loop(0, rs_per_tile)
        def _z3(j):
            for kk in range(8):
                bounce_v[j, pl.ds(kk * 16, 16)] = jnp.zeros((16,), jnp.float32)

        pltpu.sync_copy(bounce_v, rs_sh.at[pl.ds(s * rs_per_tile, rs_per_tile)])
        plsc.subcore_barrier()

        @pl.loop(0, chunks)
        def _chunk(i):
            base = base0 + i * b
            pltpu.sync_copy(src_hbm.at[pl.ds(base, b)], si_v)
            pltpu.sync_copy(e_hbm.at[pl.ds(base, b)], ev_v)

            @plsc.parallel_loop(0, b // 16, unroll=5)
            def _grp(gidx):
                evec = ev_v[pl.ds(gidx * 16, 16)]
                sivec = si_v[pl.ds(gidx * 16, 16)]
                si8_v[pl.ds(gidx * 16, 16)] = lax.shift_right_logical(sivec, 3)
                for jj in range(16):
                    j = gidx * 16 + jj
                    r = sivec[jj] & 7
                    rs_src_v[j, pl.ds(0, 16)] = jnp.where(lane == r, evec[jj], 0.0)

            pltpu.sync_copy(rs_src_v, rs_sh.at[si8_v], add=True)

        plsc.subcore_barrier()
        pltpu.sync_copy(rs_sh.at[pl.ds(s * rs_per_tile, rs_per_tile)], bounce_v)
        pltpu.sync_copy(bounce_v,
                        ors_hbm.at[pl.ds(c * rs_rows + s * rs_per_tile,
                                         rs_per_tile)])

    return k(src, edge_e), rs_rows


def _finalize(a0, a1, rs_col, n, d):
    bn = 2000
    g = n // bn
    hf = d // 2

    def body(a0_ref, a1_ref, r_ref, o_ref):
        rs = r_ref[...]
        rs = jnp.where(rs == 0.0, 1.0, rs)
        o_ref[...] = _lrelu(
            jnp.concatenate([a0_ref[...], a1_ref[...]], axis=1) / rs)

    return pl.pallas_call(
        body,
        grid=(g,),
        in_specs=[
            pl.BlockSpec((bn, hf), lambda i: (i, 0)),
            pl.BlockSpec((bn, hf), lambda i: (i, 0)),
            pl.BlockSpec((bn, 1), lambda i: (i, 0)),
        ],
        out_specs=pl.BlockSpec((bn, d), lambda i: (i, 0)),
        out_shape=jax.ShapeDtypeStruct((n, d), jnp.float32),
    )(a0, a1, rs_col)


def _bn_affine(stats, gamma, beta, count, eps=1e-5):
    mean = stats[0] / count
    var = stats[1] / count - mean * mean
    inv = gamma / jnp.sqrt(var + eps)
    scale = inv
    shift = beta - mean * inv
    return scale.reshape(1, -1), shift.reshape(1, -1)


def kernel(inputs, edge_index, w, a1_w, a1_b, bn1_g, bn1_b,
           a2_w, a2_b, bn2_g, bn2_b, a3_w, a3_b):
    e = edge_index.shape[1]
    src = edge_index[0].astype(jnp.int32)
    dst = edge_index[1].astype(jnp.int32)

    h, h0, h1 = _matmul_h(inputs, w)
    edge_h = _sc_gather_absdiff(h, src, dst)

    x1, st1 = _stage1(edge_h, a1_w)
    scale1, shift1 = _bn_affine(st1, bn1_g, bn1_b, e)
    x2, st2 = _stage2(x1, scale1, shift1, a2_w)
    scale2, shift2 = _bn_affine(st2, bn2_g, bn2_b, e)
    ep = x2.shape[0]
    pad = ep - e
    src2 = jnp.concatenate([src, jnp.zeros((pad,), jnp.int32)]).reshape(ep // 128, 128)
    dst2 = jnp.concatenate([dst, jnp.ones((pad,), jnp.int32)]).reshape(ep // 128, 128)
    edge_e = _stage3(x2, scale2, shift2, a3_w.reshape(1, 1, -1),
                     a3_b.reshape(1, 1), src2, dst2)

    n, d = inputs.shape[0], w.shape[1]
    ef = edge_e.reshape(ep)[:e]
    acc, n_pad = _sc_scatter(h0, h1, n, src, dst, ef)
    rs, rs_rows = _sc_rowsum(n, src, ef)
    a_rows = n_pad // 2
    a0 = acc[:a_rows].reshape(n_pad, d // 2)[:n]
    a1 = acc[a_rows:].reshape(n_pad, d // 2)[:n]
    rs_col = (rs[:rs_rows, :8] + rs[rs_rows:, :8]).reshape(n_pad)[:n].reshape(n, 1)
    return _finalize(a0, a1, rs_col, n, d)


# speedup vs baseline: 3.4005x; 1.1583x over previous
import functools

import jax
import jax.numpy as jnp
from jax import lax
from jax.experimental import pallas as pl
from jax.experimental.pallas import tpu as pltpu
from jax.experimental.pallas import tpu_sc as plsc

NC = 2
NS = 16
NW = NC * NS
LRELU_SLOPE = 0.2
ACC_W = 144


def _lrelu(x):
    return jnp.where(x > 0, x, LRELU_SLOPE * x)


def _matmul_h(inputs, w):
    n, d_in = inputs.shape
    d_out = w.shape[1]
    hf = d_out // 2

    def body(x_ref, w_ref, o_ref, o0_ref, o1_ref):
        o = jnp.dot(x_ref[...], w_ref[...], preferred_element_type=jnp.float32)
        o_ref[...] = o
        o0_ref[...] = o[:, :hf]
        o1_ref[...] = o[:, hf:]

    return pl.pallas_call(
        body,
        out_shape=[
            jax.ShapeDtypeStruct((n, d_out), jnp.float32),
            jax.ShapeDtypeStruct((n, hf), jnp.float32),
            jax.ShapeDtypeStruct((n, hf), jnp.float32),
        ],
    )(inputs, w)


def _sc_gather_absdiff(h, src, dst):
    n, d = h.shape
    e = src.shape[0]
    tpe = e // NW
    b = 80
    chunks = tpe // b
    nv = d // 16
    mesh = plsc.VectorSubcoreMesh(core_axis_name="c", subcore_axis_name="s")

    @functools.partial(
        pl.kernel,
        out_type=jax.ShapeDtypeStruct((e, d), jnp.float32),
        mesh=mesh,
        scratch_types=[
            pltpu.VMEM((2, b), jnp.int32),
            pltpu.VMEM((2, b), jnp.int32),
            pltpu.VMEM((2, b, d), jnp.float32),
            pltpu.VMEM((2, b, d), jnp.float32),
            [pltpu.SemaphoreType.DMA] * 2,
            [pltpu.SemaphoreType.DMA] * 2,
            [pltpu.SemaphoreType.DMA] * 2,
        ],
    )
    def k(h_hbm, src_hbm, dst_hbm, out_hbm, si_v, di_v, a_v, b_v,
          semi, semg, semo):
        wid = lax.axis_index("s") * NC + lax.axis_index("c")
        base0 = wid * tpe

        def idx_issue(j, sl):
            pltpu.async_copy(src_hbm.at[pl.ds(base0 + j * b, b)], si_v.at[sl], semi[sl])
            pltpu.async_copy(dst_hbm.at[pl.ds(base0 + j * b, b)], di_v.at[sl], semi[sl])

        def idx_wait(sl):
            pltpu.make_async_copy(src_hbm.at[pl.ds(0, b)], si_v.at[sl], semi[sl]).wait()
            pltpu.make_async_copy(dst_hbm.at[pl.ds(0, b)], di_v.at[sl], semi[sl]).wait()

        def gather_issue(sl):
            pltpu.async_copy(h_hbm.at[si_v.at[sl]], a_v.at[sl], semg[sl])
            pltpu.async_copy(h_hbm.at[di_v.at[sl]], b_v.at[sl], semg[sl])

        def gather_wait(sl):
            pltpu.make_async_copy(h_hbm.at[si_v.at[sl]], a_v.at[sl], semg[sl]).wait()
            pltpu.make_async_copy(h_hbm.at[di_v.at[sl]], b_v.at[sl], semg[sl]).wait()

        def compute(sl):
            @plsc.parallel_loop(0, b, unroll=4)
            def _row(j):
                for kk in range(nv):
                    s_ = pl.ds(kk * 16, 16)
                    a_v[sl, j, s_] = jnp.abs(a_v[sl, j, s_] - b_v[sl, j, s_])

        def out_issue(j, sl):
            pltpu.async_copy(a_v.at[sl], out_hbm.at[pl.ds(base0 + j * b, b)], semo[sl])

        def out_wait(sl):
            pltpu.make_async_copy(a_v.at[sl], out_hbm.at[pl.ds(0, b)], semo[sl]).wait()

        pltpu.sync_copy(src_hbm.at[pl.ds(base0, b)], si_v.at[0])
        pltpu.sync_copy(dst_hbm.at[pl.ds(base0, b)], di_v.at[0])
        gather_issue(0)
        idx_issue(1, 1)

        @pl.loop(0, chunks - 1, step=2)
        def _main(g):
            for bb in range(2):
                j = g + bb
                osl = 1 - bb
                idx_wait(osl)

                @pl.when(j >= 1)
                def _():
                    out_wait(osl)

                gather_issue(osl)

                @pl.when(j + 2 < chunks)
                def _():
                    idx_issue(j + 2, bb)

                gather_wait(bb)
                compute(bb)
                out_issue(j, bb)

        out_wait(1)
        gather_wait(0)
        compute(0)
        pltpu.sync_copy(a_v.at[0], out_hbm.at[pl.ds(base0 + (chunks - 1) * b, b)])

    return k(h, src, dst)


def _stage1(edge_h, a1_w):
    e, d = edge_h.shape
    f = a1_w.shape[1]
    be = 20000
    g = e // be

    def body(eh_ref, w_ref, x1_ref, st_ref):
        x1 = jnp.dot(eh_ref[...], w_ref[...], preferred_element_type=jnp.float32)
        x1_ref[...] = x1

        @pl.when(pl.program_id(0) == 0)
        def _():
            st_ref[...] = jnp.zeros_like(st_ref)

        st_ref[...] += jnp.stack(
            [jnp.sum(x1, axis=0), jnp.sum(x1 * x1, axis=0)])

    return pl.pallas_call(
        body,
        grid=(g,),
        in_specs=[
            pl.BlockSpec((be, d), lambda i: (i, 0)),
            pl.BlockSpec((d, f), lambda i: (0, 0)),
        ],
        out_specs=[
            pl.BlockSpec((be, f), lambda i: (i, 0)),
            pl.BlockSpec((2, f), lambda i: (0, 0)),
        ],
        out_shape=[
            jax.ShapeDtypeStruct((e, f), jnp.float32),
            jax.ShapeDtypeStruct((2, f), jnp.float32),
        ],
    )(edge_h, a1_w)


def _stage2(x1, scale1, shift1, a2_w):
    e, f1 = x1.shape
    f2 = a2_w.shape[1]
    be = 20000
    g = e // be
    ep = ((e + 128 * 8 * 64 - 1) // (128 * 64 * 8)) * (128 * 64 * 8)

    def body(x1_ref, sc_ref, sh_ref, w_ref, x2_ref, st_ref):
        y = _lrelu(x1_ref[...] * sc_ref[...] + sh_ref[...])
        x2 = jnp.dot(y, w_ref[...], preferred_element_type=jnp.float32)
        x2_ref[...] = x2

        @pl.when(pl.program_id(0) == 0)
        def _():
            st_ref[...] = jnp.zeros_like(st_ref)

        st_ref[...] += jnp.stack(
            [jnp.sum(x2, axis=0), jnp.sum(x2 * x2, axis=0)])

    return pl.pallas_call(
        body,
        grid=(g,),
        in_specs=[
            pl.BlockSpec((be, f1), lambda i: (i, 0)),
            pl.BlockSpec((1, f1), lambda i: (0, 0)),
            pl.BlockSpec((1, f1), lambda i: (0, 0)),
            pl.BlockSpec((f1, f2), lambda i: (0, 0)),
        ],
        out_specs=[
            pl.BlockSpec((be, f2), lambda i: (i, 0)),
            pl.BlockSpec((2, f2), lambda i: (0, 0)),
        ],
        out_shape=[
            jax.ShapeDtypeStruct((ep, f2), jnp.float32),
            jax.ShapeDtypeStruct((2, f2), jnp.float32),
        ],
    )(x1, scale1, shift1, a2_w)


def _stage3(x2p, scale2, shift2, a3_w_row, a3_b, src2, dst2):
    ep, f2 = x2p.shape
    rows = ep // 128
    br = 64
    g = rows // br
    be = br * 128

    def body(x2_ref, sc_ref, sh_ref, w3_ref, b3_ref, s_ref, d_ref, o_ref):
        y = _lrelu(x2_ref[...] * sc_ref[...] + sh_ref[...])
        t = jnp.sum(y.reshape(br, 128, f2) * w3_ref[...], axis=2) + b3_ref[0, 0]
        t = _lrelu(t)
        o_ref[...] = jnp.exp(-t) + (s_ref[...] == d_ref[...]).astype(jnp.float32)

    return pl.pallas_call(
        body,
        grid=(g,),
        in_specs=[
            pl.BlockSpec((be, f2), lambda i: (i, 0)),
            pl.BlockSpec((1, f2), lambda i: (0, 0)),
            pl.BlockSpec((1, f2), lambda i: (0, 0)),
            pl.BlockSpec((1, 1, f2), lambda i: (0, 0, 0)),
            pl.BlockSpec((1, 1), lambda i: (0, 0)),
            pl.BlockSpec((br, 128), lambda i: (i, 0)),
            pl.BlockSpec((br, 128), lambda i: (i, 0)),
        ],
        out_specs=pl.BlockSpec((br, 128), lambda i: (i, 0)),
        out_shape=jax.ShapeDtypeStruct((rows, 128), jnp.float32),
    )(x2p, scale2, shift2, a3_w_row, a3_b, src2, dst2)


def _sc_scatter(h0, h1, n, src, dst, edge_e):
    hf = h0.shape[1]
    e = src.shape[0]
    tpe = e // NS
    b = 80
    chunks = tpe // b
    nvh = hf // 16
    n_pad = ((n + 1023) // 1024) * 1024
    rows_per_tile = n_pad // NS
    mesh = plsc.VectorSubcoreMesh(core_axis_name="c", subcore_axis_name="s")

    @functools.partial(
        pl.kernel,
        out_type=jax.ShapeDtypeStruct((NC * n_pad, hf), jnp.float32),
        mesh=mesh,
        compiler_params=pltpu.CompilerParams(use_tc_tiling_on_sc=False),
        scratch_types=[
            pltpu.VMEM((2, b), jnp.int32),
            pltpu.VMEM((2, b), jnp.int32),
            pltpu.VMEM((2, b), jnp.int32),
            pltpu.VMEM((2, b), jnp.float32),
            pltpu.VMEM((2, b, hf), jnp.float32),
            pltpu.VMEM((2, b, hf), jnp.float32),
            pltpu.VMEM((rows_per_tile, hf), jnp.float32),
            pltpu.VMEM_SHARED((n_pad, hf), jnp.float32),
            [pltpu.SemaphoreType.DMA] * 2,
            [pltpu.SemaphoreType.DMA] * 2,
            [pltpu.SemaphoreType.DMA] * 2,
        ],
    )
    def k(h0_hbm, h1_hbm, src_hbm, dst_hbm, e_hbm, out_hbm,
          si_v, si2_v, di_v, ev_v, rows_v, sc_v, bounce_v, acc_sh,
          semi, semg, sems):
        c = lax.axis_index("c")
        s = lax.axis_index("s")
        base0 = s * tpe

        def idx_issue(j, sl):
            pltpu.async_copy(src_hbm.at[pl.ds(base0 + j * b, b)], si_v.at[sl], semi[sl])
            pltpu.async_copy(dst_hbm.at[pl.ds(base0 + j * b, b)], di_v.at[sl], semi[sl])
            pltpu.async_copy(e_hbm.at[pl.ds(base0 + j * b, b)], ev_v.at[sl], semi[sl])

        def idx_wait(sl):
            pltpu.make_async_copy(src_hbm.at[pl.ds(0, b)], si_v.at[sl], semi[sl]).wait()
            pltpu.make_async_copy(dst_hbm.at[pl.ds(0, b)], di_v.at[sl], semi[sl]).wait()
            pltpu.make_async_copy(e_hbm.at[pl.ds(0, b)], ev_v.at[sl], semi[sl]).wait()

        def gather_issue(sl):
            @pl.when(c == 0)
            def _g0():
                pltpu.async_copy(h0_hbm.at[di_v.at[sl]], rows_v.at[sl], semg[sl])

            @pl.when(c == 1)
            def _g1():
                pltpu.async_copy(h1_hbm.at[di_v.at[sl]], rows_v.at[sl], semg[sl])

        def gather_wait(sl):
            pltpu.make_async_copy(h0_hbm.at[di_v.at[sl]], rows_v.at[sl], semg[sl]).wait()

        def compute(sl):
            @plsc.parallel_loop(0, b // 16, unroll=5)
            def _grp(gidx):
                evec = ev_v[sl, pl.ds(gidx * 16, 16)]
                si2_v[sl, pl.ds(gidx * 16, 16)] = si_v[sl, pl.ds(gidx * 16, 16)]
                for jj in range(16):
                    j = gidx * 16 + jj
                    ee = evec[jj]
                    for kk in range(nvh):
                        sc_v[sl, j, pl.ds(kk * 16, 16)] = (
                            rows_v[sl, j, pl.ds(kk * 16, 16)] * ee)

        def scatter_issue(sl):
            pltpu.async_copy(sc_v.at[sl], acc_sh.at[si2_v.at[sl]], sems[sl], add=True)

        def scatter_wait(sl):
            pltpu.make_async_copy(sc_v.at[sl], acc_sh.at[si2_v.at[sl]], sems[sl]).wait()

        @pl.loop(0, rows_per_tile)
        def _z(j):
            for kk in range(nvh):
                bounce_v[j, pl.ds(kk * 16, 16)] = jnp.zeros((16,), jnp.float32)

        pltpu.sync_copy(bounce_v, acc_sh.at[pl.ds(s * rows_per_tile, rows_per_tile)])
        plsc.subcore_barrier()

        pltpu.sync_copy(src_hbm.at[pl.ds(base0, b)], si_v.at[0])
        pltpu.sync_copy(dst_hbm.at[pl.ds(base0, b)], di_v.at[0])
        pltpu.sync_copy(e_hbm.at[pl.ds(base0, b)], ev_v.at[0])
        gather_issue(0)
        idx_issue(1, 1)

        @pl.loop(0, chunks, step=2)
        def _main(g):
            for bb in range(2):
                j = g + bb
                osl = 1 - bb

                @pl.when(j + 1 < chunks)
                def _():
                    idx_wait(osl)
                    gather_issue(osl)

                gather_wait(bb)

                @pl.when(j >= 2)
                def _():
                    scatter_wait(bb)

                compute(bb)

                @pl.when(j + 2 < chunks)
                def _():
                    idx_issue(j + 2, bb)

                scatter_issue(bb)

        scatter_wait(0)
        scatter_wait(1)
        plsc.subcore_barrier()
        pltpu.sync_copy(acc_sh.at[pl.ds(s * rows_per_tile, rows_per_tile)], bounce_v)
        pltpu.sync_copy(bounce_v,
                        out_hbm.at[pl.ds(c * n_pad + s * rows_per_tile,
                                         rows_per_tile)])

    return k(h0, h1, src, dst, edge_e), n_pad


def _sc_rowsum(n, src, edge_e):
    e = src.shape[0]
    tpe = e // NW
    b = 80
    chunks = tpe // b
    n_pad = ((n + 1023) // 1024) * 1024
    rs_rows = n_pad // 16
    rs_per_tile = rs_rows // NS
    mesh = plsc.VectorSubcoreMesh(core_axis_name="c", subcore_axis_name="s")

    @functools.partial(
        pl.kernel,
        out_type=jax.ShapeDtypeStruct((NC * rs_rows, 128), jnp.float32),
        mesh=mesh,
        compiler_params=pltpu.CompilerParams(use_tc_tiling_on_sc=False),
        scratch_types=[
            pltpu.VMEM((2, b), jnp.int32),
            pltpu.VMEM((2, b), jnp.int32),
            pltpu.VMEM((2, b), jnp.float32),
            pltpu.VMEM((2, b, 128), jnp.float32),
            pltpu.VMEM((rs_per_tile, 128), jnp.float32),
            pltpu.VMEM_SHARED((rs_rows, 128), jnp.float32),
            [pltpu.SemaphoreType.DMA] * 2,
            [pltpu.SemaphoreType.DMA] * 2,
        ],
    )
    def k(src_hbm, e_hbm, ors_hbm, si_v, si8_v, ev_v, rs_src_v, bounce_v,
          rs_sh, semi, semr):
        c = lax.axis_index("c")
        s = lax.axis_index("s")
        wid = s * NC + c
        base0 = wid * tpe
        lane = lax.iota(jnp.int32, 16)

        def idx_issue(j, sl):
            pltpu.async_copy(src_hbm.at[pl.ds(base0 + j * b, b)], si_v.at[sl], semi[sl])
            pltpu.async_copy(e_hbm.at[pl.ds(base0 + j * b, b)], ev_v.at[sl], semi[sl])

        def idx_wait(sl):
            pltpu.make_async_copy(src_hbm.at[pl.ds(0, b)], si_v.at[sl], semi[sl]).wait()
            pltpu.make_async_copy(e_hbm.at[pl.ds(0, b)], ev_v.at[sl], semi[sl]).wait()

        def compute(sl):
            @plsc.parallel_loop(0, b // 16, unroll=5)
            def _grp(gidx):
                evec = ev_v[sl, pl.ds(gidx * 16, 16)]
                sivec = si_v[sl, pl.ds(gidx * 16, 16)]
                si8_v[sl, pl.ds(gidx * 16, 16)] = lax.shift_right_logical(sivec, 4)
                for jj in range(16):
                    r = sivec[jj] & 15
                    rs_src_v[sl, gidx * 16 + jj, pl.ds(0, 16)] = jnp.where(
                        lane == r, evec[jj], 0.0)

        def rs_issue(sl):
            pltpu.async_copy(rs_src_v.at[sl], rs_sh.at[si8_v.at[sl]], semr[sl], add=True)

        def rs_wait(sl):
            pltpu.make_async_copy(rs_src_v.at[sl], rs_sh.at[si8_v.at[sl]], semr[sl]).wait()

        @pl.loop(0, b)
        def _z(j):
            for sl in range(2):
                for kk in range(8):
                    rs_src_v[sl, j, pl.ds(kk * 16, 16)] = jnp.zeros((16,), jnp.float32)

        @pl.loop(0, rs_per_tile)
        def _z2(j):
            for kk in range(8):
                bounce_v[j, pl.ds(kk * 16, 16)] = jnp.zeros((16,), jnp.float32)

        pltpu.sync_copy(bounce_v, rs_sh.at[pl.ds(s * rs_per_tile, rs_per_tile)])
        plsc.subcore_barrier()

        pltpu.sync_copy(src_hbm.at[pl.ds(base0, b)], si_v.at[0])
        pltpu.sync_copy(e_hbm.at[pl.ds(base0, b)], ev_v.at[0])
        idx_issue(1, 1)

        @pl.loop(0, chunks - 1, step=2)
        def _main(g):
            for bb in range(2):
                j = g + bb
                osl = 1 - bb
                idx_wait(osl)

                @pl.when(j >= 2)
                def _():
                    rs_wait(bb)

                compute(bb)

                @pl.when(j + 2 < chunks)
                def _():
                    idx_issue(j + 2, bb)

                rs_issue(bb)

        rs_wait(0)
        compute(0)
        rs_issue(0)
        rs_wait(0)
        rs_wait(1)
        plsc.subcore_barrier()
        pltpu.sync_copy(rs_sh.at[pl.ds(s * rs_per_tile, rs_per_tile)], bounce_v)
        pltpu.sync_copy(bounce_v,
                        ors_hbm.at[pl.ds(c * rs_rows + s * rs_per_tile,
                                         rs_per_tile)])

    return k(src, edge_e), rs_rows


def _finalize(a0, a1, rs_col, n, d):
    bn = 2000
    g = n // bn
    hf = d // 2

    def body(a0_ref, a1_ref, r_ref, o_ref):
        rs = r_ref[...]
        rs = jnp.where(rs == 0.0, 1.0, rs)
        o_ref[...] = _lrelu(
            jnp.concatenate([a0_ref[...], a1_ref[...]], axis=1) / rs)

    return pl.pallas_call(
        body,
        grid=(g,),
        in_specs=[
            pl.BlockSpec((bn, hf), lambda i: (i, 0)),
            pl.BlockSpec((bn, hf), lambda i: (i, 0)),
            pl.BlockSpec((bn, 1), lambda i: (i, 0)),
        ],
        out_specs=pl.BlockSpec((bn, d), lambda i: (i, 0)),
        out_shape=jax.ShapeDtypeStruct((n, d), jnp.float32),
    )(a0, a1, rs_col)


def _bn_affine(stats, gamma, beta, count, eps=1e-5):
    mean = stats[0] / count
    var = stats[1] / count - mean * mean
    inv = gamma / jnp.sqrt(var + eps)
    scale = inv
    shift = beta - mean * inv
    return scale.reshape(1, -1), shift.reshape(1, -1)


def kernel(inputs, edge_index, w, a1_w, a1_b, bn1_g, bn1_b,
           a2_w, a2_b, bn2_g, bn2_b, a3_w, a3_b):
    e = edge_index.shape[1]
    src = edge_index[0].astype(jnp.int32)
    dst = edge_index[1].astype(jnp.int32)

    h, h0, h1 = _matmul_h(inputs, w)
    edge_h = _sc_gather_absdiff(h, src, dst)

    x1, st1 = _stage1(edge_h, a1_w)
    scale1, shift1 = _bn_affine(st1, bn1_g, bn1_b, e)
    x2, st2 = _stage2(x1, scale1, shift1, a2_w)
    scale2, shift2 = _bn_affine(st2, bn2_g, bn2_b, e)
    ep = x2.shape[0]
    pad = ep - e
    src2 = jnp.concatenate([src, jnp.zeros((pad,), jnp.int32)]).reshape(ep // 128, 128)
    dst2 = jnp.concatenate([dst, jnp.ones((pad,), jnp.int32)]).reshape(ep // 128, 128)
    edge_e = _stage3(x2, scale2, shift2, a3_w.reshape(1, 1, -1),
                     a3_b.reshape(1, 1), src2, dst2)

    n, d = inputs.shape[0], w.shape[1]
    ef = edge_e.reshape(ep)[:e]
    acc, n_pad = _sc_scatter(h0, h1, n, src, dst, ef)
    rs, rs_rows = _sc_rowsum(n, src, ef)
    a0 = acc[:n_pad][:n]
    a1 = acc[n_pad:][:n]
    rs_col = (rs[:rs_rows, :16] + rs[rs_rows:, :16]).reshape(n_pad)[:n].reshape(n, 1)
    return _finalize(a0, a1, rs_col, n, d)
